# baseline (speedup 1.0000x reference)
import functools

import jax
import jax.numpy as jnp
import numpy as np
from jax import lax
from jax.experimental import pallas as pl
from jax.experimental.pallas import tpu as pltpu
from jax.experimental.pallas import tpu_sc as plsc

R_CUTOFF = 5.0
NUM_RBF = 6
NUM_SBF = 7
EMBED = 128
ENV_P = 6
ANGLE_EMB = 64
OUT_EMB = 256
N_INTER = 4

N_EDGES = 160000
N_ANGLES = 320000
N_ATOMS_PAD = 10240

BE = 1000
BA = 1000
BP = 1024


def _sph_jl_np(l, x):
    x = np.asarray(x, dtype=np.float64)
    j0 = np.sin(x) / x
    if l == 0:
        return j0
    j1 = np.sin(x) / x**2 - np.cos(x) / x
    if l == 1:
        return j1
    jm, jc = j0, j1
    for i in range(1, l):
        jn = (2 * i + 1) / x * jc - jm
        jm, jc = jc, jn
    return jc


def _bessel_zeros(num_l, num_n):
    zeros = np.zeros((num_l, num_n))
    xs = np.linspace(1e-2, 80.0, 160001)
    for l in range(num_l):
        vals = _sph_jl_np(l, xs)
        s = np.sign(vals)
        idx = np.where(s[:-1] * s[1:] < 0)[0][:num_n]
        for n, i in enumerate(idx):
            a, b = xs[i], xs[i + 1]
            fa = _sph_jl_np(l, np.array([a]))[0]
            for _ in range(60):
                mid = 0.5 * (a + b)
                fm = _sph_jl_np(l, np.array([mid]))[0]
                if fa * fm <= 0:
                    b = mid
                else:
                    a, fa = mid, fm
            zeros[l, n] = 0.5 * (a + b)
    return zeros


_ZEROS_NP = _bessel_zeros(NUM_SBF, NUM_RBF)
_NORM_NP = np.zeros((NUM_SBF, NUM_RBF))
for _l in range(NUM_SBF):
    _NORM_NP[_l] = np.sqrt(2.0 / R_CUTOFF**3) / np.abs(_sph_jl_np(_l + 1, _ZEROS_NP[_l]))

_LEG_NP = np.sqrt((2 * np.arange(NUM_SBF) + 1) / (4 * np.pi)).astype(np.float32)

_ZFLAT = np.ones((1, 48), np.float32)
_ZFLAT[0, :42] = _ZEROS_NP.reshape(-1).astype(np.float32)
_NFLAT = np.zeros((1, 48), np.float32)
_NFLAT[0, :42] = _NORM_NP.reshape(-1).astype(np.float32)
_SEL = np.zeros((8, 48), np.float32)
for _l in range(NUM_SBF):
    _SEL[_l, _l * 6:(_l + 1) * 6] = 1.0
_EXP = np.zeros((8, 48), np.float32)
for _l in range(NUM_SBF):
    _EXP[_l, _l * 6:(_l + 1) * 6] = 1.0


def _swish(x):
    return x / (1.0 + jnp.exp(-x))


def _mm(a, b):
    return jnp.dot(a, b, preferred_element_type=jnp.float32)


def _wspec(shape):
    nd = len(shape)
    return pl.BlockSpec(shape, lambda i, _n=nd: (0,) * _n)


def _rspec(shape):
    return pl.BlockSpec(shape, lambda i: (i,) + (0,) * (len(shape) - 1))


def _basis_body(x_ref, zflat_ref, nflat_ref, sel_ref, freq_ref, rbf_ref, rad_ref):
    x = x_ref[...]
    arg = x * zflat_ref[...]
    inv = 1.0 / x
    x2 = x * x
    x3 = x2 * x
    x6 = x3 * x3
    x7 = x6 * x
    x8 = x6 * x2
    p = ENV_P + 1
    a = -(p + 1) * (p + 2) / 2.0
    b = p * (p + 2)
    c = -p * (p + 1) / 2.0
    env = jnp.where(x < 1.0, inv + a * x6 + b * x7 + c * x8, 0.0)

    s = jnp.sin(arg)
    co = jnp.cos(arg)
    j0 = s / arg
    j1 = s / (arg * arg) - co / arg
    js = [j0, j1]
    for i in range(1, NUM_SBF - 1):
        js.append((2 * i + 1) / arg * js[i] - js[i - 1])
    sel = jnp.zeros_like(arg)
    for l in range(NUM_SBF):
        sel = sel + js[l] * sel_ref[l:l + 1, :]
    rad_ref[...] = sel * nflat_ref[...] * env
    rbf_ref[...] = env * jnp.sin(freq_ref[...] * x)


def _basis(x_col, zflat, nflat, sel, freq8):
    n = x_col.shape[0]
    return pl.pallas_call(
        _basis_body,
        grid=(n // BE,),
        in_specs=[_rspec((BE, 1)), _wspec((1, 48)), _wspec((1, 48)),
                  _wspec((8, 48)), _wspec((1, 8))],
        out_specs=[_rspec((BE, 8)), _rspec((BE, 48))],
        out_shape=[jax.ShapeDtypeStruct((n, 8), jnp.float32),
                   jax.ShapeDtypeStruct((n, 48), jnp.float32)],
    )(x_col, zflat, nflat, sel, freq8)


def _cbf_body(ang_ref, mask_ref, out_ref):
    ct = jnp.cos(ang_ref[...])
    ps = [jnp.ones_like(ct), ct]
    for l in range(1, NUM_SBF - 1):
        ps.append(((2 * l + 1) * ct * ps[l] - l * ps[l - 1]) / (l + 1))
    msk = mask_ref[...]
    cols = [_LEG_NP[l] * ps[l] * msk for l in range(NUM_SBF)]
    cols.append(jnp.zeros_like(ct))
    out_ref[...] = jnp.concatenate(cols, axis=1)


def _cbf(ang_col, mask_col):
    n = ang_col.shape[0]
    return pl.pallas_call(
        _cbf_body,
        grid=(n // BA,),
        in_specs=[_rspec((BA, 1)), _rspec((BA, 1))],
        out_specs=_rspec((BA, 8)),
        out_shape=jax.ShapeDtypeStruct((n, 8), jnp.float32),
    )(ang_col, mask_col)


def _sbf48_body(radg_ref, cbf_ref, exp_ref, out_ref):
    out_ref[...] = radg_ref[...] * _mm(cbf_ref[...], exp_ref[...])


def _sbf48(rad_g, cbf8, exp):
    n = rad_g.shape[0]
    return pl.pallas_call(
        _sbf48_body,
        grid=(n // BA,),
        in_specs=[_rspec((BA, 48)), _rspec((BA, 8)), _wspec((8, 48))],
        out_specs=_rspec((BA, 48)),
        out_shape=jax.ShapeDtypeStruct((n, 48), jnp.float32),
    )(rad_g, cbf8, exp)


def _edge_embed_body(hj_ref, hi_ref, rbf_ref, wre_ref, bre_ref, wj_ref,
                     wi_ref, wr_ref, be_ref, wrbf0_ref, m_ref, prod_ref):
    rbf = rbf_ref[...]
    rbf_e = _swish(_mm(rbf, wre_ref[...]) + bre_ref[...])
    m = _swish(_mm(hj_ref[...], wj_ref[...]) + _mm(hi_ref[...], wi_ref[...])
               + _mm(rbf_e, wr_ref[...]) + be_ref[...])
    m_ref[...] = m
    prod_ref[...] = _mm(rbf, wrbf0_ref[...]) * m


def _edge_embed(hj, hi, rbf8, wre, bre, wj, wi, wr, be_, wrbf0):
    n = hj.shape[0]
    return pl.pallas_call(
        _edge_embed_body,
        grid=(n // BE,),
        in_specs=[_rspec((BE, 64)), _rspec((BE, 64)), _rspec((BE, 8)),
                  _wspec((8, 128)), _wspec((1, 128)), _wspec((64, 128)),
                  _wspec((64, 128)), _wspec((128, 128)), _wspec((1, 128)),
                  _wspec((8, 128))],
        out_specs=[_rspec((BE, 128)), _rspec((BE, 128))],
        out_shape=[jax.ShapeDtypeStruct((n, 128), jnp.float32),
                   jax.ShapeDtypeStruct((n, 128), jnp.float32)],
    )(hj, hi, rbf8, wre, bre, wj, wi, wr, be_, wrbf0)


def _inter_a_body(m_ref, rbf_ref, wji_ref, bji_ref, wkj_ref, bkj_ref,
                  wrbfc_ref, wdown_ref, xji_ref, xdown_ref):
    m = m_ref[...]
    xji_ref[...] = _swish(_mm(m, wji_ref[...]) + bji_ref[...])
    x_kj = _swish(_mm(m, wkj_ref[...]) + bkj_ref[...])
    x_kj = x_kj * _mm(rbf_ref[...], wrbfc_ref[...])
    xdown_ref[...] = _swish(_mm(x_kj, wdown_ref[...]))


def _inter_a(m, rbf8, wji, bji, wkj, bkj, wrbfc, wdown):
    n = m.shape[0]
    return pl.pallas_call(
        _inter_a_body,
        grid=(n // BE,),
        in_specs=[_rspec((BE, 128)), _rspec((BE, 8)), _wspec((128, 128)),
                  _wspec((1, 128)), _wspec((128, 128)), _wspec((1, 128)),
                  _wspec((8, 128)), _wspec((128, 64))],
        out_specs=[_rspec((BE, 128)), _rspec((BE, 64))],
        out_shape=[jax.ShapeDtypeStruct((n, 128), jnp.float32),
                   jax.ShapeDtypeStruct((n, 64), jnp.float32)],
    )(m, rbf8, wji, bji, wkj, bkj, wrbfc, wdown)


def _tri_t_body(gx_ref, sbf_ref, wc_ref, out_ref):
    out_ref[...] = gx_ref[...] * _mm(sbf_ref[...], wc_ref[...])


def _tri_t(gx, sbf48, wc):
    n = gx.shape[0]
    return pl.pallas_call(
        _tri_t_body,
        grid=(n // BA,),
        in_specs=[_rspec((BA, 64)), _rspec((BA, 48)), _wspec((48, 64))],
        out_specs=_rspec((BA, 64)),
        out_shape=jax.ShapeDtypeStruct((n, 64), jnp.float32),
    )(gx, sbf48, wc)


def _inter_b_body(agg_ref, xji_ref, m_ref, rbf_ref, wup_ref,
                  rb_w1, rb_b1, rb_w2, rb_b2, wskip_ref, bskip_ref,
                  ra1_w1, ra1_b1, ra1_w2, ra1_b2,
                  ra2_w1, ra2_b1, ra2_w2, ra2_b2, wrbfn_ref,
                  mnew_ref, prod_ref):
    x_kj = _swish(_mm(agg_ref[...], wup_ref[...]))
    h = xji_ref[...] + x_kj
    h = h + _swish(_mm(_swish(_mm(h, rb_w1[...]) + rb_b1[...]), rb_w2[...]) + rb_b2[...])
    h = _swish(_mm(h, wskip_ref[...]) + bskip_ref[...]) + m_ref[...]
    h = h + _swish(_mm(_swish(_mm(h, ra1_w1[...]) + ra1_b1[...]), ra1_w2[...]) + ra1_b2[...])
    h = h + _swish(_mm(_swish(_mm(h, ra2_w1[...]) + ra2_b1[...]), ra2_w2[...]) + ra2_b2[...])
    mnew_ref[...] = h
    prod_ref[...] = _mm(rbf_ref[...], wrbfn_ref[...]) * h


def _inter_b(agg, xji, m, rbf8, wup, rb, wskip, bskip, ra1, ra2, wrbfn):
    n = m.shape[0]
    w128 = _wspec((128, 128))
    b128 = _wspec((1, 128))
    return pl.pallas_call(
        _inter_b_body,
        grid=(n // BE,),
        in_specs=[_rspec((BE, 64)), _rspec((BE, 128)), _rspec((BE, 128)),
                  _rspec((BE, 8)), _wspec((64, 128)),
                  w128, b128, w128, b128, w128, b128,
                  w128, b128, w128, b128,
                  w128, b128, w128, b128, _wspec((8, 128))],
        out_specs=[_rspec((BE, 128)), _rspec((BE, 128))],
        out_shape=[jax.ShapeDtypeStruct((n, 128), jnp.float32),
                   jax.ShapeDtypeStruct((n, 128), jnp.float32)],
    )(agg, xji, m, rbf8, wup,
      rb['W1'], rb['b1'].reshape(1, -1), rb['W2'], rb['b2'].reshape(1, -1),
      wskip, bskip,
      ra1['W1'], ra1['b1'].reshape(1, -1), ra1['W2'], ra1['b2'].reshape(1, -1),
      ra2['W1'], ra2['b1'].reshape(1, -1), ra2['W2'], ra2['b2'].reshape(1, -1),
      wrbfn)


def _out_mlp_body(p0_ref, p1_ref, prev_ref, wup_ref, w1, b1, w2, b2, w3, b3,
                  wout_ref, out_ref):
    t = _mm(p0_ref[...] + p1_ref[...], wup_ref[...])
    t = _swish(_mm(t, w1[...]) + b1[...])
    t = _swish(_mm(t, w2[...]) + b2[...])
    t = _swish(_mm(t, w3[...]) + b3[...])
    out_ref[...] = prev_ref[...] + _mm(t, wout_ref[...])


def _out_mlp(p0, p1, prev, ob):
    n = p0.shape[0]
    wout8 = jnp.zeros((OUT_EMB, 8), jnp.float32).at[:, :1].set(ob['W_out'])
    w256 = _wspec((256, 256))
    b256 = _wspec((1, 256))
    return pl.pallas_call(
        _out_mlp_body,
        grid=(n // BP,),
        in_specs=[_rspec((BP, 128)), _rspec((BP, 128)), _rspec((BP, 8)),
                  _wspec((128, 256)), w256, b256, w256, b256, w256, b256,
                  _wspec((256, 8))],
        out_specs=_rspec((BP, 8)),
        out_shape=jax.ShapeDtypeStruct((n, 8), jnp.float32),
    )(p0, p1, prev, ob['W_up'],
      ob['Ws'][0], ob['bs'][0].reshape(1, -1),
      ob['Ws'][1], ob['bs'][1].reshape(1, -1),
      ob['Ws'][2], ob['bs'][2].reshape(1, -1),
      wout8)


_NW = 32


@functools.lru_cache(maxsize=None)
def _make_sc_gather(V, D, B):
    assert D % 16 == 0 and B % (8 * _NW) == 0
    b_per_w = B // _NW
    nb, rem = divmod(b_per_w, 128)
    mesh = plsc.VectorSubcoreMesh(core_axis_name="c", subcore_axis_name="s")

    @functools.partial(
        pl.kernel, mesh=mesh,
        out_type=jax.ShapeDtypeStruct((B, D), jnp.float32),
        compiler_params=pltpu.CompilerParams(use_tc_tiling_on_sc=False),
        scratch_types=[
            pltpu.VMEM((b_per_w,), jnp.int32),
            pltpu.VMEM((128, D), jnp.float32),
            pltpu.SemaphoreType.DMA,
        ],
    )
    def k(table_hbm, idx_hbm, out_hbm, idx_v, rows_v, sem):
        wid = lax.axis_index("s") * 2 + lax.axis_index("c")
        base = wid * b_per_w
        pltpu.sync_copy(idx_hbm.at[pl.ds(base, b_per_w)], idx_v)

        def body(j, _):
            off = j * 128
            pltpu.async_copy(
                table_hbm.at[idx_v.at[pl.ds(off, 128)]], rows_v, sem).wait()
            pltpu.sync_copy(rows_v, out_hbm.at[pl.ds(base + off, 128)])
            return _

        lax.fori_loop(0, nb, body, 0)
        if rem:
            off = nb * 128
            pltpu.async_copy(
                table_hbm.at[idx_v.at[pl.ds(off, rem)]],
                rows_v.at[pl.ds(0, rem)], sem).wait()
            pltpu.sync_copy(rows_v.at[pl.ds(0, rem)],
                            out_hbm.at[pl.ds(base + off, rem)])

    return k


def _gather_rows(table, idx):
    V, D = table.shape
    B = idx.shape[0]
    return _make_sc_gather(V, D, B)(table, idx)


def _segsum(rows, idx, nseg):
    return jax.ops.segment_sum(rows, idx, num_segments=nseg)


@functools.lru_cache(maxsize=None)
def _make_sc_segsum_atoms(E, D, NSEG):
    assert E % _NW == 0 and NSEG % 16 == 0
    e_per_w = E // _NW
    nb, rem = divmod(e_per_w, 128)
    stripe = NSEG // 16
    mesh = plsc.VectorSubcoreMesh(core_axis_name="c", subcore_axis_name="s")

    @functools.partial(
        pl.kernel, mesh=mesh,
        out_type=jax.ShapeDtypeStruct((2 * NSEG, D), jnp.float32),
        compiler_params=pltpu.CompilerParams(use_tc_tiling_on_sc=False),
        scratch_types=[
            pltpu.VMEM_SHARED((NSEG, D), jnp.float32),
            pltpu.VMEM((128,), jnp.int32),
            pltpu.VMEM((8,), jnp.int32),
            pltpu.VMEM((128, D), jnp.float32),
            pltpu.SemaphoreType.DMA,
        ],
    )
    def k(src_hbm, idx_hbm, zero_hbm, out_hbm, shared, idx_v, idxt_v, rows_v, sem):
        c = lax.axis_index("c")
        s = lax.axis_index("s")
        wid = s * 2 + c
        base = wid * e_per_w
        pltpu.sync_copy(zero_hbm, shared.at[pl.ds(s * stripe, stripe)])
        plsc.subcore_barrier()

        def body(j, _):
            off = base + j * 128
            pltpu.sync_copy(idx_hbm.at[pl.ds(off, 128)], idx_v)
            pltpu.sync_copy(src_hbm.at[pl.ds(off, 128)], rows_v)
            pltpu.sync_copy(rows_v, shared.at[idx_v], add=True)
            return _

        lax.fori_loop(0, nb, body, 0)
        if rem:
            off = base + nb * 128
            pltpu.sync_copy(idx_hbm.at[pl.ds(off, rem)], idxt_v)
            pltpu.sync_copy(src_hbm.at[pl.ds(off, rem)], rows_v.at[pl.ds(0, rem)])
            pltpu.sync_copy(rows_v.at[pl.ds(0, rem)], shared.at[idxt_v], add=True)
        plsc.subcore_barrier()
        pltpu.sync_copy(shared.at[pl.ds(s * stripe, stripe)],
                        out_hbm.at[pl.ds(c * NSEG + s * stripe, stripe)])

    return k


_TRI_R = 16000
_TRI_NRANGE = 10


_RING = 4
_HALF = 2


@functools.lru_cache(maxsize=None)
def _make_sc_segsum_tri(A, D, NSEG):
    assert A % 16 == 0 and NSEG == _TRI_R * _TRI_NRANGE
    a_per_t = A // 16
    nb, rem = divmod(a_per_t, 128)
    assert nb % _RING == 0 and rem % 16 == 0
    ngrp = nb // _RING
    zstripe = (_TRI_R + 16) // 16
    ostripe = _TRI_R // 16
    mesh = plsc.VectorSubcoreMesh(core_axis_name="c", subcore_axis_name="s")

    rows_scratch = [pltpu.VMEM((128, D), jnp.float32) for _ in range(_RING)]
    loc_scratch = [pltpu.VMEM((128,), jnp.int32) for _ in range(_RING)]
    sem_scratch = [pltpu.SemaphoreType.DMA for _ in range(2 * _RING)]

    @functools.partial(
        pl.kernel, mesh=mesh,
        out_type=jax.ShapeDtypeStruct((NSEG, D), jnp.float32),
        compiler_params=pltpu.CompilerParams(use_tc_tiling_on_sc=False),
        scratch_types=[
            pltpu.VMEM_SHARED((_TRI_R + 16, D), jnp.float32),
            pltpu.VMEM((a_per_t,), jnp.int32),
            pltpu.VMEM((32,), jnp.int32),
        ] + rows_scratch + loc_scratch + sem_scratch,
    )
    def k(src_hbm, idx_hbm, zero_hbm, out_hbm, shared, idx_v, loct_v, *bufs):
        rows = bufs[:_RING]
        locs = bufs[_RING:2 * _RING]
        ldsem = bufs[2 * _RING:3 * _RING]
        scsem = bufs[3 * _RING:4 * _RING]
        c = lax.axis_index("c")
        s = lax.axis_index("s")
        tbase = s * a_per_t
        pltpu.sync_copy(idx_hbm.at[pl.ds(tbase, a_per_t)], idx_v)

        def localize(j, n, dst, rng_base):
            for v in range(n // 16):
                iv = idx_v[pl.ds(j * 128 + v * 16, 16)]
                loc = iv - rng_base
                oob = (loc < 0) | (loc >= _TRI_R)
                dst[pl.ds(v * 16, 16)] = jnp.where(oob, _TRI_R, loc)

        def start_load(j, b):
            pltpu.async_copy(src_hbm.at[pl.ds(tbase + j * 128, 128)],
                             rows[b], ldsem[b])

        def wait_load(b):
            pltpu.make_async_copy(src_hbm.at[pl.ds(0, 128)], rows[b],
                                  ldsem[b]).wait()

        for pr in range(_TRI_NRANGE // 2):
            rng_base = (c * (_TRI_NRANGE // 2) + pr) * _TRI_R
            pltpu.sync_copy(zero_hbm, shared.at[pl.ds(s * zstripe, zstripe)])
            plsc.subcore_barrier()

            for b in range(_HALF):
                localize(b, 128, locs[b], rng_base)
                start_load(b, b)

            def grp(g, carry):
                for b in range(_RING):
                    j = g * _RING + b
                    wait_load(b)
                    pltpu.async_copy(rows[b], shared.at[locs[b]], scsem[b],
                                     add=True)
                    bn = (b + _HALF) % _RING
                    jn = j + _HALF

                    @pl.when(jn >= _RING)
                    def _():
                        pltpu.make_async_copy(
                            rows[bn], shared.at[locs[bn]], scsem[bn]).wait()

                    @pl.when(jn < nb)
                    def _():
                        localize(jn, 128, locs[bn], rng_base)
                        start_load(jn, bn)
                return carry

            lax.fori_loop(0, ngrp, grp, 0)
            for b in range(_HALF, _RING):
                pltpu.make_async_copy(rows[b], shared.at[locs[b]],
                                      scsem[b]).wait()
            if rem:
                localize(nb, rem, loct_v, rng_base)
                pltpu.sync_copy(src_hbm.at[pl.ds(tbase + nb * 128, rem)],
                                rows[0].at[pl.ds(0, rem)])
                pltpu.sync_copy(rows[0].at[pl.ds(0, rem)],
                                shared.at[loct_v], add=True)
            plsc.subcore_barrier()
            pltpu.sync_copy(shared.at[pl.ds(s * ostripe, ostripe)],
                            out_hbm.at[pl.ds(rng_base + s * ostripe, ostripe)])
            plsc.subcore_barrier()

    return k


def _pad8(w, rows=8):
    out = jnp.zeros((rows,) + w.shape[1:], w.dtype)
    return out.at[:w.shape[0]].set(w)


@jax.jit
def _forward(distances, angles, params, species, idx_i, idx_j, angle_mask,
             reduce_to_ji, expand_to_kj):
    zflat = jnp.asarray(_ZFLAT)
    nflat = jnp.asarray(_NFLAT)
    sel = jnp.asarray(_SEL)
    exp = jnp.asarray(_EXP)
    freq8 = _pad8(params['freq'].reshape(-1, 1), 8).reshape(1, 8)

    x_col = (distances / R_CUTOFF).reshape(-1, 1)
    rbf8, rad48 = _basis(x_col, zflat, nflat, sel, freq8)

    cbf8 = _cbf(angles.reshape(-1, 1),
                angle_mask.astype(jnp.float32).reshape(-1, 1))

    expand_i32 = expand_to_kj.astype(jnp.int32)
    rad_g = _gather_rows(rad48, expand_i32)
    sbf = _sbf48(rad_g, cbf8, exp)

    species_p = jnp.zeros((N_ATOMS_PAD,), jnp.int32).at[:species.shape[0]].set(
        species.astype(jnp.int32))
    h = _gather_rows(params['emb'], species_p)
    hj = _gather_rows(h, idx_j.astype(jnp.int32))
    hi = _gather_rows(h, idx_i.astype(jnp.int32))

    we = params['W_edge']
    m, prod = _edge_embed(
        hj, hi, rbf8,
        _pad8(params['W_rbf_emb']), params['b_rbf_emb'].reshape(1, -1),
        we[:64], we[64:128], we[128:], params['b_edge'].reshape(1, -1),
        _pad8(params['out_blocks'][0]['W_rbf']))

    out_acc = jnp.zeros((N_ATOMS_PAD, 8), jnp.float32)
    idx_i32 = idx_i.astype(jnp.int32)
    reduce_i32 = reduce_to_ji.astype(jnp.int32)
    z_atoms = jnp.zeros((N_ATOMS_PAD // 16, 128), jnp.float32)
    z_tri = jnp.zeros(((_TRI_R + 16) // 16, 64), jnp.float32)
    seg_atoms = _make_sc_segsum_atoms(N_EDGES, 128, N_ATOMS_PAD)
    seg_tri = _make_sc_segsum_tri(N_ANGLES, 64, N_EDGES)

    for i in range(N_INTER + 1):
        pf = seg_atoms(prod, idx_i32, z_atoms)
        out_acc = _out_mlp(pf[:N_ATOMS_PAD], pf[N_ATOMS_PAD:], out_acc,
                           params['out_blocks'][i])
        if i == N_INTER:
            break
        ip = params['int_blocks'][i]
        wrbfc = _pad8(_mm(ip['W_rbf1'], ip['W_rbf2']))
        wc48 = _pad8(_mm(ip['W_sbf1'], ip['W_sbf2']), 48)
        xji, xdown = _inter_a(m, rbf8, ip['W_ji'], ip['b_ji'].reshape(1, -1),
                              ip['W_kj'], ip['b_kj'].reshape(1, -1),
                              wrbfc, ip['W_down'])
        gx = _gather_rows(xdown, expand_i32)
        t = _tri_t(gx, sbf, wc48)
        agg = seg_tri(t, reduce_i32, z_tri)
        m, prod = _inter_b(agg, xji, m, rbf8, ip['W_up'],
                           ip['res_before'][0], ip['W_skip'],
                           ip['b_skip'].reshape(1, -1),
                           ip['res_after'][0], ip['res_after'][1],
                           _pad8(params['out_blocks'][i + 1]['W_rbf']))

    return out_acc[:10000, :1]


def kernel(distances, angles, params, species, idx_i, idx_j, angle_mask,
           reduce_to_ji, expand_to_kj):
    return _forward(distances, angles, params, species, idx_i, idx_j,
                    angle_mask, reduce_to_ji, expand_to_kj)


# speedup vs baseline: 1.1050x; 1.1050x over previous
import functools

import jax
import jax.numpy as jnp
import numpy as np
from jax import lax
from jax.experimental import pallas as pl
from jax.experimental.pallas import tpu as pltpu
from jax.experimental.pallas import tpu_sc as plsc

R_CUTOFF = 5.0
NUM_RBF = 6
NUM_SBF = 7
EMBED = 128
ENV_P = 6
ANGLE_EMB = 64
OUT_EMB = 256
N_INTER = 4

N_EDGES = 160000
N_ANGLES = 320000
N_ATOMS_PAD = 10240

BE = 1000
BA = 1000
BP = 1024


def _sph_jl_np(l, x):
    x = np.asarray(x, dtype=np.float64)
    j0 = np.sin(x) / x
    if l == 0:
        return j0
    j1 = np.sin(x) / x**2 - np.cos(x) / x
    if l == 1:
        return j1
    jm, jc = j0, j1
    for i in range(1, l):
        jn = (2 * i + 1) / x * jc - jm
        jm, jc = jc, jn
    return jc


def _bessel_zeros(num_l, num_n):
    zeros = np.zeros((num_l, num_n))
    xs = np.linspace(1e-2, 80.0, 160001)
    for l in range(num_l):
        vals = _sph_jl_np(l, xs)
        s = np.sign(vals)
        idx = np.where(s[:-1] * s[1:] < 0)[0][:num_n]
        for n, i in enumerate(idx):
            a, b = xs[i], xs[i + 1]
            fa = _sph_jl_np(l, np.array([a]))[0]
            for _ in range(60):
                mid = 0.5 * (a + b)
                fm = _sph_jl_np(l, np.array([mid]))[0]
                if fa * fm <= 0:
                    b = mid
                else:
                    a, fa = mid, fm
            zeros[l, n] = 0.5 * (a + b)
    return zeros


_ZEROS_NP = _bessel_zeros(NUM_SBF, NUM_RBF)
_NORM_NP = np.zeros((NUM_SBF, NUM_RBF))
for _l in range(NUM_SBF):
    _NORM_NP[_l] = np.sqrt(2.0 / R_CUTOFF**3) / np.abs(_sph_jl_np(_l + 1, _ZEROS_NP[_l]))

_LEG_NP = np.sqrt((2 * np.arange(NUM_SBF) + 1) / (4 * np.pi)).astype(np.float32)

_ZFLAT = np.ones((1, 48), np.float32)
_ZFLAT[0, :42] = _ZEROS_NP.reshape(-1).astype(np.float32)
_NFLAT = np.zeros((1, 48), np.float32)
_NFLAT[0, :42] = _NORM_NP.reshape(-1).astype(np.float32)
_SEL = np.zeros((8, 48), np.float32)
for _l in range(NUM_SBF):
    _SEL[_l, _l * 6:(_l + 1) * 6] = 1.0
_EXP = np.zeros((8, 48), np.float32)
for _l in range(NUM_SBF):
    _EXP[_l, _l * 6:(_l + 1) * 6] = 1.0


def _swish(x):
    return x / (1.0 + jnp.exp(-x))


def _mm(a, b):
    return jnp.dot(a, b, preferred_element_type=jnp.float32)


def _wspec(shape):
    nd = len(shape)
    return pl.BlockSpec(shape, lambda i, _n=nd: (0,) * _n)


def _rspec(shape):
    return pl.BlockSpec(shape, lambda i: (i,) + (0,) * (len(shape) - 1))


def _basis_body(x_ref, zflat_ref, nflat_ref, sel_ref, freq_ref, rbf_ref, rad_ref):
    x = x_ref[...]
    arg = x * zflat_ref[...]
    inv = 1.0 / x
    x2 = x * x
    x3 = x2 * x
    x6 = x3 * x3
    x7 = x6 * x
    x8 = x6 * x2
    p = ENV_P + 1
    a = -(p + 1) * (p + 2) / 2.0
    b = p * (p + 2)
    c = -p * (p + 1) / 2.0
    env = jnp.where(x < 1.0, inv + a * x6 + b * x7 + c * x8, 0.0)

    s = jnp.sin(arg)
    co = jnp.cos(arg)
    j0 = s / arg
    j1 = s / (arg * arg) - co / arg
    js = [j0, j1]
    for i in range(1, NUM_SBF - 1):
        js.append((2 * i + 1) / arg * js[i] - js[i - 1])
    sel = jnp.zeros_like(arg)
    for l in range(NUM_SBF):
        sel = sel + js[l] * sel_ref[l:l + 1, :]
    rad_ref[...] = sel * nflat_ref[...] * env
    rbf_ref[...] = env * jnp.sin(freq_ref[...] * x)


def _basis(x_col, zflat, nflat, sel, freq8):
    n = x_col.shape[0]
    return pl.pallas_call(
        _basis_body,
        grid=(n // BE,),
        in_specs=[_rspec((BE, 1)), _wspec((1, 48)), _wspec((1, 48)),
                  _wspec((8, 48)), _wspec((1, 8))],
        out_specs=[_rspec((BE, 8)), _rspec((BE, 48))],
        out_shape=[jax.ShapeDtypeStruct((n, 8), jnp.float32),
                   jax.ShapeDtypeStruct((n, 48), jnp.float32)],
    )(x_col, zflat, nflat, sel, freq8)


def _cbf_body(ang_ref, mask_ref, out_ref):
    ct = jnp.cos(ang_ref[...])
    ps = [jnp.ones_like(ct), ct]
    for l in range(1, NUM_SBF - 1):
        ps.append(((2 * l + 1) * ct * ps[l] - l * ps[l - 1]) / (l + 1))
    msk = mask_ref[...]
    cols = [_LEG_NP[l] * ps[l] * msk for l in range(NUM_SBF)]
    cols.append(jnp.zeros_like(ct))
    out_ref[...] = jnp.concatenate(cols, axis=1)


def _cbf(ang_col, mask_col):
    n = ang_col.shape[0]
    return pl.pallas_call(
        _cbf_body,
        grid=(n // BA,),
        in_specs=[_rspec((BA, 1)), _rspec((BA, 1))],
        out_specs=_rspec((BA, 8)),
        out_shape=jax.ShapeDtypeStruct((n, 8), jnp.float32),
    )(ang_col, mask_col)


def _sbf48_body(radg_ref, cbf_ref, exp_ref, out_ref):
    out_ref[...] = radg_ref[...] * _mm(cbf_ref[...], exp_ref[...])


def _sbf48(rad_g, cbf8, exp):
    n = rad_g.shape[0]
    return pl.pallas_call(
        _sbf48_body,
        grid=(n // BA,),
        in_specs=[_rspec((BA, 48)), _rspec((BA, 8)), _wspec((8, 48))],
        out_specs=_rspec((BA, 48)),
        out_shape=jax.ShapeDtypeStruct((n, 48), jnp.float32),
    )(rad_g, cbf8, exp)


def _edge_embed_body(hj_ref, hi_ref, rbf_ref, wre_ref, bre_ref, wj_ref,
                     wi_ref, wr_ref, be_ref, wrbf0_ref, m_ref, prod_ref):
    rbf = rbf_ref[...]
    rbf_e = _swish(_mm(rbf, wre_ref[...]) + bre_ref[...])
    m = _swish(_mm(hj_ref[...], wj_ref[...]) + _mm(hi_ref[...], wi_ref[...])
               + _mm(rbf_e, wr_ref[...]) + be_ref[...])
    m_ref[...] = m
    prod_ref[...] = _mm(rbf, wrbf0_ref[...]) * m


def _edge_embed(hj, hi, rbf8, wre, bre, wj, wi, wr, be_, wrbf0):
    n = hj.shape[0]
    return pl.pallas_call(
        _edge_embed_body,
        grid=(n // BE,),
        in_specs=[_rspec((BE, 64)), _rspec((BE, 64)), _rspec((BE, 8)),
                  _wspec((8, 128)), _wspec((1, 128)), _wspec((64, 128)),
                  _wspec((64, 128)), _wspec((128, 128)), _wspec((1, 128)),
                  _wspec((8, 128))],
        out_specs=[_rspec((BE, 128)), _rspec((BE, 128))],
        out_shape=[jax.ShapeDtypeStruct((n, 128), jnp.float32),
                   jax.ShapeDtypeStruct((n, 128), jnp.float32)],
    )(hj, hi, rbf8, wre, bre, wj, wi, wr, be_, wrbf0)


def _inter_a_body(m_ref, rbf_ref, wji_ref, bji_ref, wkj_ref, bkj_ref,
                  wrbfc_ref, wdown_ref, xji_ref, xdown_ref):
    m = m_ref[...]
    xji_ref[...] = _swish(_mm(m, wji_ref[...]) + bji_ref[...])
    x_kj = _swish(_mm(m, wkj_ref[...]) + bkj_ref[...])
    x_kj = x_kj * _mm(rbf_ref[...], wrbfc_ref[...])
    xdown_ref[...] = _swish(_mm(x_kj, wdown_ref[...]))


def _inter_a(m, rbf8, wji, bji, wkj, bkj, wrbfc, wdown):
    n = m.shape[0]
    return pl.pallas_call(
        _inter_a_body,
        grid=(n // BE,),
        in_specs=[_rspec((BE, 128)), _rspec((BE, 8)), _wspec((128, 128)),
                  _wspec((1, 128)), _wspec((128, 128)), _wspec((1, 128)),
                  _wspec((8, 128)), _wspec((128, 64))],
        out_specs=[_rspec((BE, 128)), _rspec((BE, 64))],
        out_shape=[jax.ShapeDtypeStruct((n, 128), jnp.float32),
                   jax.ShapeDtypeStruct((n, 64), jnp.float32)],
    )(m, rbf8, wji, bji, wkj, bkj, wrbfc, wdown)


def _tri_t_body(gx_ref, sbf_ref, wc_ref, out_ref):
    out_ref[...] = gx_ref[...] * _mm(sbf_ref[...], wc_ref[...])


def _tri_t(gx, sbf48, wc):
    n = gx.shape[0]
    return pl.pallas_call(
        _tri_t_body,
        grid=(n // BA,),
        in_specs=[_rspec((BA, 64)), _rspec((BA, 48)), _wspec((48, 64))],
        out_specs=_rspec((BA, 64)),
        out_shape=jax.ShapeDtypeStruct((n, 64), jnp.float32),
    )(gx, sbf48, wc)


def _inter_b_body(agg_ref, xji_ref, m_ref, rbf_ref, wup_ref,
                  rb_w1, rb_b1, rb_w2, rb_b2, wskip_ref, bskip_ref,
                  ra1_w1, ra1_b1, ra1_w2, ra1_b2,
                  ra2_w1, ra2_b1, ra2_w2, ra2_b2, wrbfn_ref,
                  mnew_ref, prod_ref):
    x_kj = _swish(_mm(agg_ref[...], wup_ref[...]))
    h = xji_ref[...] + x_kj
    h = h + _swish(_mm(_swish(_mm(h, rb_w1[...]) + rb_b1[...]), rb_w2[...]) + rb_b2[...])
    h = _swish(_mm(h, wskip_ref[...]) + bskip_ref[...]) + m_ref[...]
    h = h + _swish(_mm(_swish(_mm(h, ra1_w1[...]) + ra1_b1[...]), ra1_w2[...]) + ra1_b2[...])
    h = h + _swish(_mm(_swish(_mm(h, ra2_w1[...]) + ra2_b1[...]), ra2_w2[...]) + ra2_b2[...])
    mnew_ref[...] = h
    prod_ref[...] = _mm(rbf_ref[...], wrbfn_ref[...]) * h


def _inter_b(agg, xji, m, rbf8, wup, rb, wskip, bskip, ra1, ra2, wrbfn):
    n = m.shape[0]
    w128 = _wspec((128, 128))
    b128 = _wspec((1, 128))
    return pl.pallas_call(
        _inter_b_body,
        grid=(n // BE,),
        in_specs=[_rspec((BE, 64)), _rspec((BE, 128)), _rspec((BE, 128)),
                  _rspec((BE, 8)), _wspec((64, 128)),
                  w128, b128, w128, b128, w128, b128,
                  w128, b128, w128, b128,
                  w128, b128, w128, b128, _wspec((8, 128))],
        out_specs=[_rspec((BE, 128)), _rspec((BE, 128))],
        out_shape=[jax.ShapeDtypeStruct((n, 128), jnp.float32),
                   jax.ShapeDtypeStruct((n, 128), jnp.float32)],
    )(agg, xji, m, rbf8, wup,
      rb['W1'], rb['b1'].reshape(1, -1), rb['W2'], rb['b2'].reshape(1, -1),
      wskip, bskip,
      ra1['W1'], ra1['b1'].reshape(1, -1), ra1['W2'], ra1['b2'].reshape(1, -1),
      ra2['W1'], ra2['b1'].reshape(1, -1), ra2['W2'], ra2['b2'].reshape(1, -1),
      wrbfn)


def _out_mlp_body(p0_ref, p1_ref, prev_ref, wup_ref, w1, b1, w2, b2, w3, b3,
                  wout_ref, out_ref):
    t = _mm(p0_ref[...] + p1_ref[...], wup_ref[...])
    t = _swish(_mm(t, w1[...]) + b1[...])
    t = _swish(_mm(t, w2[...]) + b2[...])
    t = _swish(_mm(t, w3[...]) + b3[...])
    out_ref[...] = prev_ref[...] + _mm(t, wout_ref[...])


def _out_mlp(p0, p1, prev, ob):
    n = p0.shape[0]
    wout8 = jnp.zeros((OUT_EMB, 8), jnp.float32).at[:, :1].set(ob['W_out'])
    w256 = _wspec((256, 256))
    b256 = _wspec((1, 256))
    return pl.pallas_call(
        _out_mlp_body,
        grid=(n // BP,),
        in_specs=[_rspec((BP, 128)), _rspec((BP, 128)), _rspec((BP, 8)),
                  _wspec((128, 256)), w256, b256, w256, b256, w256, b256,
                  _wspec((256, 8))],
        out_specs=_rspec((BP, 8)),
        out_shape=jax.ShapeDtypeStruct((n, 8), jnp.float32),
    )(p0, p1, prev, ob['W_up'],
      ob['Ws'][0], ob['bs'][0].reshape(1, -1),
      ob['Ws'][1], ob['bs'][1].reshape(1, -1),
      ob['Ws'][2], ob['bs'][2].reshape(1, -1),
      wout8)


_NW = 32


@functools.lru_cache(maxsize=None)
def _make_sc_gather(V, D, B):
    assert D % 16 == 0 and B % (8 * _NW) == 0
    b_per_w = B // _NW
    nb, rem = divmod(b_per_w, 128)
    mesh = plsc.VectorSubcoreMesh(core_axis_name="c", subcore_axis_name="s")

    @functools.partial(
        pl.kernel, mesh=mesh,
        out_type=jax.ShapeDtypeStruct((B, D), jnp.float32),
        compiler_params=pltpu.CompilerParams(use_tc_tiling_on_sc=False),
        scratch_types=[
            pltpu.VMEM((b_per_w,), jnp.int32),
            pltpu.VMEM((128, D), jnp.float32),
            pltpu.SemaphoreType.DMA,
        ],
    )
    def k(table_hbm, idx_hbm, out_hbm, idx_v, rows_v, sem):
        wid = lax.axis_index("s") * 2 + lax.axis_index("c")
        base = wid * b_per_w
        pltpu.sync_copy(idx_hbm.at[pl.ds(base, b_per_w)], idx_v)

        def body(j, _):
            off = j * 128
            pltpu.async_copy(
                table_hbm.at[idx_v.at[pl.ds(off, 128)]], rows_v, sem).wait()
            pltpu.sync_copy(rows_v, out_hbm.at[pl.ds(base + off, 128)])
            return _

        lax.fori_loop(0, nb, body, 0)
        if rem:
            off = nb * 128
            pltpu.async_copy(
                table_hbm.at[idx_v.at[pl.ds(off, rem)]],
                rows_v.at[pl.ds(0, rem)], sem).wait()
            pltpu.sync_copy(rows_v.at[pl.ds(0, rem)],
                            out_hbm.at[pl.ds(base + off, rem)])

    return k


def _gather_rows(table, idx):
    V, D = table.shape
    B = idx.shape[0]
    return _make_sc_gather(V, D, B)(table, idx)


def _segsum(rows, idx, nseg):
    return jax.ops.segment_sum(rows, idx, num_segments=nseg)


@functools.lru_cache(maxsize=None)
def _make_sc_segsum_atoms(E, D, NSEG):
    assert E % _NW == 0 and NSEG % 16 == 0
    e_per_w = E // _NW
    nb, rem = divmod(e_per_w, 128)
    stripe = NSEG // 16
    mesh = plsc.VectorSubcoreMesh(core_axis_name="c", subcore_axis_name="s")

    @functools.partial(
        pl.kernel, mesh=mesh,
        out_type=jax.ShapeDtypeStruct((2 * NSEG, D), jnp.float32),
        compiler_params=pltpu.CompilerParams(use_tc_tiling_on_sc=False),
        scratch_types=[
            pltpu.VMEM_SHARED((NSEG, D), jnp.float32),
            pltpu.VMEM((128,), jnp.int32),
            pltpu.VMEM((8,), jnp.int32),
            pltpu.VMEM((128, D), jnp.float32),
            pltpu.SemaphoreType.DMA,
        ],
    )
    def k(src_hbm, idx_hbm, zero_hbm, out_hbm, shared, idx_v, idxt_v, rows_v, sem):
        c = lax.axis_index("c")
        s = lax.axis_index("s")
        wid = s * 2 + c
        base = wid * e_per_w
        pltpu.sync_copy(zero_hbm, shared.at[pl.ds(s * stripe, stripe)])
        plsc.subcore_barrier()

        def body(j, _):
            off = base + j * 128
            pltpu.sync_copy(idx_hbm.at[pl.ds(off, 128)], idx_v)
            pltpu.sync_copy(src_hbm.at[pl.ds(off, 128)], rows_v)
            pltpu.sync_copy(rows_v, shared.at[idx_v], add=True)
            return _

        lax.fori_loop(0, nb, body, 0)
        if rem:
            off = base + nb * 128
            pltpu.sync_copy(idx_hbm.at[pl.ds(off, rem)], idxt_v)
            pltpu.sync_copy(src_hbm.at[pl.ds(off, rem)], rows_v.at[pl.ds(0, rem)])
            pltpu.sync_copy(rows_v.at[pl.ds(0, rem)], shared.at[idxt_v], add=True)
        plsc.subcore_barrier()
        pltpu.sync_copy(shared.at[pl.ds(s * stripe, stripe)],
                        out_hbm.at[pl.ds(c * NSEG + s * stripe, stripe)])

    return k


_TRI_R = 20000
_TRI_NRANGE = 8


_SB = 512
_NSUB = _SB // 128


@functools.lru_cache(maxsize=None)
def _make_sc_segsum_tri(A, D, NSEG):
    assert A % 16 == 0 and NSEG == _TRI_R * _TRI_NRANGE
    a_per_t = A // 16
    nb, rem = divmod(a_per_t, _SB)
    assert rem % 16 == 0 and rem <= 128
    zstripe = (_TRI_R + 16) // 16
    ostripe = _TRI_R // 16
    mesh = plsc.VectorSubcoreMesh(core_axis_name="c", subcore_axis_name="s")

    loc_scratch = [pltpu.VMEM((128,), jnp.int32) for _ in range(_NSUB)]

    @functools.partial(
        pl.kernel, mesh=mesh,
        out_type=jax.ShapeDtypeStruct((NSEG, D), jnp.float32),
        compiler_params=pltpu.CompilerParams(use_tc_tiling_on_sc=False),
        scratch_types=[
            pltpu.VMEM_SHARED((_TRI_R + 16, D), jnp.float32),
            pltpu.VMEM((_SB,), jnp.int32),
            pltpu.VMEM((32,), jnp.int32),
            pltpu.VMEM((_SB, D), jnp.float32),
        ] + loc_scratch,
    )
    def k(src_hbm, idx_hbm, zero_hbm, out_hbm, shared, idx_v, loct_v, rows_v,
          *locs):
        c = lax.axis_index("c")
        s = lax.axis_index("s")
        tbase = s * a_per_t

        def localize(n, dst, voff, rng_base):
            for v in range(n // 16):
                iv = idx_v[pl.ds(voff * 128 + v * 16, 16)]
                loc = iv - rng_base
                oob = (loc < 0) | (loc >= _TRI_R)
                dst[pl.ds(v * 16, 16)] = jnp.where(oob, _TRI_R, loc)

        for pr in range(_TRI_NRANGE // 2):
            rng_base = (c * (_TRI_NRANGE // 2) + pr) * _TRI_R
            pltpu.sync_copy(zero_hbm, shared.at[pl.ds(s * zstripe, zstripe)])
            plsc.subcore_barrier()

            def body(g, carry):
                off = tbase + g * _SB
                pltpu.sync_copy(idx_hbm.at[pl.ds(off, _SB)], idx_v)
                pltpu.sync_copy(src_hbm.at[pl.ds(off, _SB)], rows_v)
                for q in range(_NSUB):
                    localize(128, locs[q], q, rng_base)
                    pltpu.sync_copy(rows_v.at[pl.ds(q * 128, 128)],
                                    shared.at[locs[q]], add=True)
                return carry

            lax.fori_loop(0, nb, body, 0)
            if rem:
                off = tbase + nb * _SB
                pltpu.sync_copy(idx_hbm.at[pl.ds(off, rem)],
                                idx_v.at[pl.ds(0, rem)])
                localize(rem, loct_v, 0, rng_base)
                pltpu.sync_copy(src_hbm.at[pl.ds(off, rem)],
                                rows_v.at[pl.ds(0, rem)])
                pltpu.sync_copy(rows_v.at[pl.ds(0, rem)],
                                shared.at[loct_v], add=True)
            plsc.subcore_barrier()
            pltpu.sync_copy(shared.at[pl.ds(s * ostripe, ostripe)],
                            out_hbm.at[pl.ds(rng_base + s * ostripe, ostripe)])
            plsc.subcore_barrier()

    return k


def _pad8(w, rows=8):
    out = jnp.zeros((rows,) + w.shape[1:], w.dtype)
    return out.at[:w.shape[0]].set(w)


@jax.jit
def _forward(distances, angles, params, species, idx_i, idx_j, angle_mask,
             reduce_to_ji, expand_to_kj):
    zflat = jnp.asarray(_ZFLAT)
    nflat = jnp.asarray(_NFLAT)
    sel = jnp.asarray(_SEL)
    exp = jnp.asarray(_EXP)
    freq8 = _pad8(params['freq'].reshape(-1, 1), 8).reshape(1, 8)

    x_col = (distances / R_CUTOFF).reshape(-1, 1)
    rbf8, rad48 = _basis(x_col, zflat, nflat, sel, freq8)

    cbf8 = _cbf(angles.reshape(-1, 1),
                angle_mask.astype(jnp.float32).reshape(-1, 1))

    expand_i32 = expand_to_kj.astype(jnp.int32)
    rad_g = _gather_rows(rad48, expand_i32)
    sbf = _sbf48(rad_g, cbf8, exp)

    species_p = jnp.zeros((N_ATOMS_PAD,), jnp.int32).at[:species.shape[0]].set(
        species.astype(jnp.int32))
    h = _gather_rows(params['emb'], species_p)
    hj = _gather_rows(h, idx_j.astype(jnp.int32))
    hi = _gather_rows(h, idx_i.astype(jnp.int32))

    we = params['W_edge']
    m, prod = _edge_embed(
        hj, hi, rbf8,
        _pad8(params['W_rbf_emb']), params['b_rbf_emb'].reshape(1, -1),
        we[:64], we[64:128], we[128:], params['b_edge'].reshape(1, -1),
        _pad8(params['out_blocks'][0]['W_rbf']))

    out_acc = jnp.zeros((N_ATOMS_PAD, 8), jnp.float32)
    idx_i32 = idx_i.astype(jnp.int32)
    reduce_i32 = reduce_to_ji.astype(jnp.int32)
    z_atoms = jnp.zeros((N_ATOMS_PAD // 16, 128), jnp.float32)
    z_tri = jnp.zeros(((_TRI_R + 16) // 16, 64), jnp.float32)
    seg_atoms = _make_sc_segsum_atoms(N_EDGES, 128, N_ATOMS_PAD)
    seg_tri = _make_sc_segsum_tri(N_ANGLES, 64, N_EDGES)

    for i in range(N_INTER + 1):
        pf = seg_atoms(prod, idx_i32, z_atoms)
        out_acc = _out_mlp(pf[:N_ATOMS_PAD], pf[N_ATOMS_PAD:], out_acc,
                           params['out_blocks'][i])
        if i == N_INTER:
            break
        ip = params['int_blocks'][i]
        wrbfc = _pad8(_mm(ip['W_rbf1'], ip['W_rbf2']))
        wc48 = _pad8(_mm(ip['W_sbf1'], ip['W_sbf2']), 48)
        xji, xdown = _inter_a(m, rbf8, ip['W_ji'], ip['b_ji'].reshape(1, -1),
                              ip['W_kj'], ip['b_kj'].reshape(1, -1),
                              wrbfc, ip['W_down'])
        gx = _gather_rows(xdown, expand_i32)
        t = _tri_t(gx, sbf, wc48)
        agg = seg_tri(t, reduce_i32, z_tri)
        m, prod = _inter_b(agg, xji, m, rbf8, ip['W_up'],
                           ip['res_before'][0], ip['W_skip'],
                           ip['b_skip'].reshape(1, -1),
                           ip['res_after'][0], ip['res_after'][1],
                           _pad8(params['out_blocks'][i + 1]['W_rbf']))

    return out_acc[:10000, :1]


def kernel(distances, angles, params, species, idx_i, idx_j, angle_mask,
           reduce_to_ji, expand_to_kj):
    return _forward(distances, angles, params, species, idx_i, idx_j,
                    angle_mask, reduce_to_ji, expand_to_kj)


# speedup vs baseline: 1.3880x; 1.2561x over previous
import functools

import jax
import jax.numpy as jnp
import numpy as np
from jax import lax
from jax.experimental import pallas as pl
from jax.experimental.pallas import tpu as pltpu
from jax.experimental.pallas import tpu_sc as plsc

R_CUTOFF = 5.0
NUM_RBF = 6
NUM_SBF = 7
EMBED = 128
ENV_P = 6
ANGLE_EMB = 64
OUT_EMB = 256
N_INTER = 4

N_EDGES = 160000
N_ANGLES = 320000
N_ATOMS_PAD = 10240

BE = 1000
BA = 1000
BP = 1024


def _sph_jl_np(l, x):
    x = np.asarray(x, dtype=np.float64)
    j0 = np.sin(x) / x
    if l == 0:
        return j0
    j1 = np.sin(x) / x**2 - np.cos(x) / x
    if l == 1:
        return j1
    jm, jc = j0, j1
    for i in range(1, l):
        jn = (2 * i + 1) / x * jc - jm
        jm, jc = jc, jn
    return jc


def _bessel_zeros(num_l, num_n):
    zeros = np.zeros((num_l, num_n))
    xs = np.linspace(1e-2, 80.0, 160001)
    for l in range(num_l):
        vals = _sph_jl_np(l, xs)
        s = np.sign(vals)
        idx = np.where(s[:-1] * s[1:] < 0)[0][:num_n]
        for n, i in enumerate(idx):
            a, b = xs[i], xs[i + 1]
            fa = _sph_jl_np(l, np.array([a]))[0]
            for _ in range(60):
                mid = 0.5 * (a + b)
                fm = _sph_jl_np(l, np.array([mid]))[0]
                if fa * fm <= 0:
                    b = mid
                else:
                    a, fa = mid, fm
            zeros[l, n] = 0.5 * (a + b)
    return zeros


_ZEROS_NP = _bessel_zeros(NUM_SBF, NUM_RBF)
_NORM_NP = np.zeros((NUM_SBF, NUM_RBF))
for _l in range(NUM_SBF):
    _NORM_NP[_l] = np.sqrt(2.0 / R_CUTOFF**3) / np.abs(_sph_jl_np(_l + 1, _ZEROS_NP[_l]))

_LEG_NP = np.sqrt((2 * np.arange(NUM_SBF) + 1) / (4 * np.pi)).astype(np.float32)

_ZFLAT = np.ones((1, 48), np.float32)
_ZFLAT[0, :42] = _ZEROS_NP.reshape(-1).astype(np.float32)
_NFLAT = np.zeros((1, 48), np.float32)
_NFLAT[0, :42] = _NORM_NP.reshape(-1).astype(np.float32)
_SEL = np.zeros((8, 48), np.float32)
for _l in range(NUM_SBF):
    _SEL[_l, _l * 6:(_l + 1) * 6] = 1.0
_EXP = np.zeros((8, 48), np.float32)
for _l in range(NUM_SBF):
    _EXP[_l, _l * 6:(_l + 1) * 6] = 1.0


def _swish(x):
    return x / (1.0 + jnp.exp(-x))


def _mm(a, b):
    return jnp.dot(a, b, preferred_element_type=jnp.float32)


def _wspec(shape):
    nd = len(shape)
    return pl.BlockSpec(shape, lambda i, _n=nd: (0,) * _n)


def _rspec(shape):
    return pl.BlockSpec(shape, lambda i: (i,) + (0,) * (len(shape) - 1))


def _basis_body(x_ref, zflat_ref, nflat_ref, sel_ref, freq_ref, rbf_ref, rad_ref):
    x = x_ref[...]
    arg = x * zflat_ref[...]
    inv = 1.0 / x
    x2 = x * x
    x3 = x2 * x
    x6 = x3 * x3
    x7 = x6 * x
    x8 = x6 * x2
    p = ENV_P + 1
    a = -(p + 1) * (p + 2) / 2.0
    b = p * (p + 2)
    c = -p * (p + 1) / 2.0
    env = jnp.where(x < 1.0, inv + a * x6 + b * x7 + c * x8, 0.0)

    s = jnp.sin(arg)
    co = jnp.cos(arg)
    j0 = s / arg
    j1 = s / (arg * arg) - co / arg
    js = [j0, j1]
    for i in range(1, NUM_SBF - 1):
        js.append((2 * i + 1) / arg * js[i] - js[i - 1])
    sel = jnp.zeros_like(arg)
    for l in range(NUM_SBF):
        sel = sel + js[l] * sel_ref[l:l + 1, :]
    rad_ref[...] = sel * nflat_ref[...] * env
    rbf_ref[...] = env * jnp.sin(freq_ref[...] * x)


def _basis(x_col, zflat, nflat, sel, freq8):
    n = x_col.shape[0]
    return pl.pallas_call(
        _basis_body,
        grid=(n // BE,),
        in_specs=[_rspec((BE, 1)), _wspec((1, 48)), _wspec((1, 48)),
                  _wspec((8, 48)), _wspec((1, 8))],
        out_specs=[_rspec((BE, 8)), _rspec((BE, 48))],
        out_shape=[jax.ShapeDtypeStruct((n, 8), jnp.float32),
                   jax.ShapeDtypeStruct((n, 48), jnp.float32)],
    )(x_col, zflat, nflat, sel, freq8)


def _cbf_body(ang_ref, mask_ref, out_ref):
    ct = jnp.cos(ang_ref[...])
    ps = [jnp.ones_like(ct), ct]
    for l in range(1, NUM_SBF - 1):
        ps.append(((2 * l + 1) * ct * ps[l] - l * ps[l - 1]) / (l + 1))
    msk = mask_ref[...]
    cols = [_LEG_NP[l] * ps[l] * msk for l in range(NUM_SBF)]
    cols.append(jnp.zeros_like(ct))
    out_ref[...] = jnp.concatenate(cols, axis=1)


def _cbf(ang_col, mask_col):
    n = ang_col.shape[0]
    return pl.pallas_call(
        _cbf_body,
        grid=(n // BA,),
        in_specs=[_rspec((BA, 1)), _rspec((BA, 1))],
        out_specs=_rspec((BA, 8)),
        out_shape=jax.ShapeDtypeStruct((n, 8), jnp.float32),
    )(ang_col, mask_col)


def _sbf48_body(radg_ref, cbf_ref, exp_ref, out_ref):
    out_ref[...] = radg_ref[...] * _mm(cbf_ref[...], exp_ref[...])


def _sbf48(rad_g, cbf8, exp):
    n = rad_g.shape[0]
    return pl.pallas_call(
        _sbf48_body,
        grid=(n // BA,),
        in_specs=[_rspec((BA, 48)), _rspec((BA, 8)), _wspec((8, 48))],
        out_specs=_rspec((BA, 48)),
        out_shape=jax.ShapeDtypeStruct((n, 48), jnp.float32),
    )(rad_g, cbf8, exp)


def _edge_embed_body(hj_ref, hi_ref, rbf_ref, wre_ref, bre_ref, wj_ref,
                     wi_ref, wr_ref, be_ref, wrbf0_ref, m_ref, prod_ref):
    rbf = rbf_ref[...]
    rbf_e = _swish(_mm(rbf, wre_ref[...]) + bre_ref[...])
    m = _swish(_mm(hj_ref[...], wj_ref[...]) + _mm(hi_ref[...], wi_ref[...])
               + _mm(rbf_e, wr_ref[...]) + be_ref[...])
    m_ref[...] = m
    prod_ref[...] = _mm(rbf, wrbf0_ref[...]) * m


def _edge_embed(hj, hi, rbf8, wre, bre, wj, wi, wr, be_, wrbf0):
    n = hj.shape[0]
    return pl.pallas_call(
        _edge_embed_body,
        grid=(n // BE,),
        in_specs=[_rspec((BE, 64)), _rspec((BE, 64)), _rspec((BE, 8)),
                  _wspec((8, 128)), _wspec((1, 128)), _wspec((64, 128)),
                  _wspec((64, 128)), _wspec((128, 128)), _wspec((1, 128)),
                  _wspec((8, 128))],
        out_specs=[_rspec((BE, 128)), _rspec((BE, 128))],
        out_shape=[jax.ShapeDtypeStruct((n, 128), jnp.float32),
                   jax.ShapeDtypeStruct((n, 128), jnp.float32)],
    )(hj, hi, rbf8, wre, bre, wj, wi, wr, be_, wrbf0)


def _inter_a_body(m_ref, rbf_ref, wji_ref, bji_ref, wkj_ref, bkj_ref,
                  wrbfc_ref, wdown_ref, xji_ref, xdown_ref):
    m = m_ref[...]
    xji_ref[...] = _swish(_mm(m, wji_ref[...]) + bji_ref[...])
    x_kj = _swish(_mm(m, wkj_ref[...]) + bkj_ref[...])
    x_kj = x_kj * _mm(rbf_ref[...], wrbfc_ref[...])
    xdown_ref[...] = _swish(_mm(x_kj, wdown_ref[...]))


def _inter_a(m, rbf8, wji, bji, wkj, bkj, wrbfc, wdown):
    n = m.shape[0]
    return pl.pallas_call(
        _inter_a_body,
        grid=(n // BE,),
        in_specs=[_rspec((BE, 128)), _rspec((BE, 8)), _wspec((128, 128)),
                  _wspec((1, 128)), _wspec((128, 128)), _wspec((1, 128)),
                  _wspec((8, 128)), _wspec((128, 64))],
        out_specs=[_rspec((BE, 128)), _rspec((BE, 64))],
        out_shape=[jax.ShapeDtypeStruct((n, 128), jnp.float32),
                   jax.ShapeDtypeStruct((n, 64), jnp.float32)],
    )(m, rbf8, wji, bji, wkj, bkj, wrbfc, wdown)


def _tri_t_body(gx_ref, sbf_ref, wc_ref, out_ref):
    out_ref[...] = gx_ref[...] * _mm(sbf_ref[...], wc_ref[...])


def _tri_t(gx, sbf48, wc):
    n = gx.shape[0]
    return pl.pallas_call(
        _tri_t_body,
        grid=(n // BA,),
        in_specs=[_rspec((BA, 64)), _rspec((BA, 48)), _wspec((48, 64))],
        out_specs=_rspec((BA, 64)),
        out_shape=jax.ShapeDtypeStruct((n, 64), jnp.float32),
    )(gx, sbf48, wc)


def _inter_b_body(agg_ref, xji_ref, m_ref, rbf_ref, wup_ref,
                  rb_w1, rb_b1, rb_w2, rb_b2, wskip_ref, bskip_ref,
                  ra1_w1, ra1_b1, ra1_w2, ra1_b2,
                  ra2_w1, ra2_b1, ra2_w2, ra2_b2, wrbfn_ref,
                  mnew_ref, prod_ref):
    x_kj = _swish(_mm(agg_ref[...], wup_ref[...]))
    h = xji_ref[...] + x_kj
    h = h + _swish(_mm(_swish(_mm(h, rb_w1[...]) + rb_b1[...]), rb_w2[...]) + rb_b2[...])
    h = _swish(_mm(h, wskip_ref[...]) + bskip_ref[...]) + m_ref[...]
    h = h + _swish(_mm(_swish(_mm(h, ra1_w1[...]) + ra1_b1[...]), ra1_w2[...]) + ra1_b2[...])
    h = h + _swish(_mm(_swish(_mm(h, ra2_w1[...]) + ra2_b1[...]), ra2_w2[...]) + ra2_b2[...])
    mnew_ref[...] = h
    prod_ref[...] = _mm(rbf_ref[...], wrbfn_ref[...]) * h


def _inter_b(agg, xji, m, rbf8, wup, rb, wskip, bskip, ra1, ra2, wrbfn):
    n = m.shape[0]
    w128 = _wspec((128, 128))
    b128 = _wspec((1, 128))
    return pl.pallas_call(
        _inter_b_body,
        grid=(n // BE,),
        in_specs=[_rspec((BE, 64)), _rspec((BE, 128)), _rspec((BE, 128)),
                  _rspec((BE, 8)), _wspec((64, 128)),
                  w128, b128, w128, b128, w128, b128,
                  w128, b128, w128, b128,
                  w128, b128, w128, b128, _wspec((8, 128))],
        out_specs=[_rspec((BE, 128)), _rspec((BE, 128))],
        out_shape=[jax.ShapeDtypeStruct((n, 128), jnp.float32),
                   jax.ShapeDtypeStruct((n, 128), jnp.float32)],
    )(agg, xji, m, rbf8, wup,
      rb['W1'], rb['b1'].reshape(1, -1), rb['W2'], rb['b2'].reshape(1, -1),
      wskip, bskip,
      ra1['W1'], ra1['b1'].reshape(1, -1), ra1['W2'], ra1['b2'].reshape(1, -1),
      ra2['W1'], ra2['b1'].reshape(1, -1), ra2['W2'], ra2['b2'].reshape(1, -1),
      wrbfn)


def _out_mlp_body(p0_ref, p1_ref, prev_ref, wup_ref, w1, b1, w2, b2, w3, b3,
                  wout_ref, out_ref):
    t = _mm(p0_ref[...] + p1_ref[...], wup_ref[...])
    t = _swish(_mm(t, w1[...]) + b1[...])
    t = _swish(_mm(t, w2[...]) + b2[...])
    t = _swish(_mm(t, w3[...]) + b3[...])
    out_ref[...] = prev_ref[...] + _mm(t, wout_ref[...])


def _out_mlp(p0, p1, prev, ob):
    n = p0.shape[0]
    wout8 = jnp.zeros((OUT_EMB, 8), jnp.float32).at[:, :1].set(ob['W_out'])
    w256 = _wspec((256, 256))
    b256 = _wspec((1, 256))
    return pl.pallas_call(
        _out_mlp_body,
        grid=(n // BP,),
        in_specs=[_rspec((BP, 128)), _rspec((BP, 128)), _rspec((BP, 8)),
                  _wspec((128, 256)), w256, b256, w256, b256, w256, b256,
                  _wspec((256, 8))],
        out_specs=_rspec((BP, 8)),
        out_shape=jax.ShapeDtypeStruct((n, 8), jnp.float32),
    )(p0, p1, prev, ob['W_up'],
      ob['Ws'][0], ob['bs'][0].reshape(1, -1),
      ob['Ws'][1], ob['bs'][1].reshape(1, -1),
      ob['Ws'][2], ob['bs'][2].reshape(1, -1),
      wout8)


_NW = 32


@functools.lru_cache(maxsize=None)
def _make_sc_gather(V, D, B):
    assert D % 16 == 0 and B % (8 * _NW) == 0
    b_per_w = B // _NW
    nb, rem = divmod(b_per_w, 128)
    mesh = plsc.VectorSubcoreMesh(core_axis_name="c", subcore_axis_name="s")

    @functools.partial(
        pl.kernel, mesh=mesh,
        out_type=jax.ShapeDtypeStruct((B, D), jnp.float32),
        compiler_params=pltpu.CompilerParams(use_tc_tiling_on_sc=False),
        scratch_types=[
            pltpu.VMEM((b_per_w,), jnp.int32),
            pltpu.VMEM((128, D), jnp.float32),
            pltpu.SemaphoreType.DMA,
        ],
    )
    def k(table_hbm, idx_hbm, out_hbm, idx_v, rows_v, sem):
        wid = lax.axis_index("s") * 2 + lax.axis_index("c")
        base = wid * b_per_w
        pltpu.sync_copy(idx_hbm.at[pl.ds(base, b_per_w)], idx_v)

        def body(j, _):
            off = j * 128
            pltpu.async_copy(
                table_hbm.at[idx_v.at[pl.ds(off, 128)]], rows_v, sem).wait()
            pltpu.sync_copy(rows_v, out_hbm.at[pl.ds(base + off, 128)])
            return _

        lax.fori_loop(0, nb, body, 0)
        if rem:
            off = nb * 128
            pltpu.async_copy(
                table_hbm.at[idx_v.at[pl.ds(off, rem)]],
                rows_v.at[pl.ds(0, rem)], sem).wait()
            pltpu.sync_copy(rows_v.at[pl.ds(0, rem)],
                            out_hbm.at[pl.ds(base + off, rem)])

    return k


def _gather_rows(table, idx):
    V, D = table.shape
    B = idx.shape[0]
    return _make_sc_gather(V, D, B)(table, idx)


def _segsum(rows, idx, nseg):
    return jax.ops.segment_sum(rows, idx, num_segments=nseg)


@functools.lru_cache(maxsize=None)
def _make_sc_segsum_atoms(E, D, NSEG):
    assert E % _NW == 0 and NSEG % 16 == 0
    e_per_w = E // _NW
    nb, rem = divmod(e_per_w, 128)
    stripe = NSEG // 16
    mesh = plsc.VectorSubcoreMesh(core_axis_name="c", subcore_axis_name="s")

    @functools.partial(
        pl.kernel, mesh=mesh,
        out_type=jax.ShapeDtypeStruct((2 * NSEG, D), jnp.float32),
        compiler_params=pltpu.CompilerParams(use_tc_tiling_on_sc=False),
        scratch_types=[
            pltpu.VMEM_SHARED((NSEG, D), jnp.float32),
            pltpu.VMEM((128,), jnp.int32),
            pltpu.VMEM((8,), jnp.int32),
            pltpu.VMEM((128, D), jnp.float32),
            pltpu.SemaphoreType.DMA,
        ],
    )
    def k(src_hbm, idx_hbm, zero_hbm, out_hbm, shared, idx_v, idxt_v, rows_v, sem):
        c = lax.axis_index("c")
        s = lax.axis_index("s")
        wid = s * 2 + c
        base = wid * e_per_w
        pltpu.sync_copy(zero_hbm, shared.at[pl.ds(s * stripe, stripe)])
        plsc.subcore_barrier()

        def body(j, _):
            off = base + j * 128
            pltpu.sync_copy(idx_hbm.at[pl.ds(off, 128)], idx_v)
            pltpu.sync_copy(src_hbm.at[pl.ds(off, 128)], rows_v)
            pltpu.sync_copy(rows_v, shared.at[idx_v], add=True)
            return _

        lax.fori_loop(0, nb, body, 0)
        if rem:
            off = base + nb * 128
            pltpu.sync_copy(idx_hbm.at[pl.ds(off, rem)], idxt_v)
            pltpu.sync_copy(src_hbm.at[pl.ds(off, rem)], rows_v.at[pl.ds(0, rem)])
            pltpu.sync_copy(rows_v.at[pl.ds(0, rem)], shared.at[idxt_v], add=True)
        plsc.subcore_barrier()
        pltpu.sync_copy(shared.at[pl.ds(s * stripe, stripe)],
                        out_hbm.at[pl.ds(c * NSEG + s * stripe, stripe)])

    return k


_TRI_R = 20000
_TRI_NRANGE = 8


_SB = 512
_NSUB = _SB // 128


@functools.lru_cache(maxsize=None)
def _make_sc_segsum_tri(A, D, NSEG):
    assert A % 16 == 0 and NSEG == _TRI_R * _TRI_NRANGE
    a_per_t = A // 16
    nb, rem = divmod(a_per_t, _SB)
    assert rem % 16 == 0 and rem <= 128
    zstripe = (_TRI_R + 16) // 16
    ostripe = _TRI_R // 16
    mesh = plsc.VectorSubcoreMesh(core_axis_name="c", subcore_axis_name="s")

    loc_scratch = [pltpu.VMEM((128,), jnp.int32) for _ in range(_NSUB)]

    @functools.partial(
        pl.kernel, mesh=mesh,
        out_type=jax.ShapeDtypeStruct((NSEG, D), jnp.float32),
        compiler_params=pltpu.CompilerParams(use_tc_tiling_on_sc=False),
        scratch_types=[
            pltpu.VMEM_SHARED((_TRI_R + 16, D), jnp.float32),
            pltpu.VMEM((_SB,), jnp.int32),
            pltpu.VMEM((32,), jnp.int32),
            pltpu.VMEM((_SB, D), jnp.float32),
        ] + loc_scratch,
    )
    def k(src_hbm, idx_hbm, zero_hbm, out_hbm, shared, idx_v, loct_v, rows_v,
          *locs):
        c = lax.axis_index("c")
        s = lax.axis_index("s")
        tbase = s * a_per_t

        dump = _TRI_R + s

        def localize(n, dst, voff, rng_base):
            for v in range(n // 16):
                iv = idx_v[pl.ds(voff * 128 + v * 16, 16)]
                loc = iv - rng_base
                oob = (loc < 0) | (loc >= _TRI_R)
                dst[pl.ds(v * 16, 16)] = jnp.where(oob, dump, loc)

        for pr in range(_TRI_NRANGE // 2):
            rng_base = (c * (_TRI_NRANGE // 2) + pr) * _TRI_R
            pltpu.sync_copy(zero_hbm, shared.at[pl.ds(s * zstripe, zstripe)])
            plsc.subcore_barrier()

            def body(g, carry):
                off = tbase + g * _SB
                pltpu.sync_copy(idx_hbm.at[pl.ds(off, _SB)], idx_v)
                pltpu.sync_copy(src_hbm.at[pl.ds(off, _SB)], rows_v)
                for q in range(_NSUB):
                    localize(128, locs[q], q, rng_base)
                    pltpu.sync_copy(rows_v.at[pl.ds(q * 128, 128)],
                                    shared.at[locs[q]], add=True)
                return carry

            lax.fori_loop(0, nb, body, 0)
            if rem:
                off = tbase + nb * _SB
                pltpu.sync_copy(idx_hbm.at[pl.ds(off, rem)],
                                idx_v.at[pl.ds(0, rem)])
                localize(rem, loct_v, 0, rng_base)
                pltpu.sync_copy(src_hbm.at[pl.ds(off, rem)],
                                rows_v.at[pl.ds(0, rem)])
                pltpu.sync_copy(rows_v.at[pl.ds(0, rem)],
                                shared.at[loct_v], add=True)
            plsc.subcore_barrier()
            pltpu.sync_copy(shared.at[pl.ds(s * ostripe, ostripe)],
                            out_hbm.at[pl.ds(rng_base + s * ostripe, ostripe)])
            plsc.subcore_barrier()

    return k


def _pad8(w, rows=8):
    out = jnp.zeros((rows,) + w.shape[1:], w.dtype)
    return out.at[:w.shape[0]].set(w)


@jax.jit
def _forward(distances, angles, params, species, idx_i, idx_j, angle_mask,
             reduce_to_ji, expand_to_kj):
    zflat = jnp.asarray(_ZFLAT)
    nflat = jnp.asarray(_NFLAT)
    sel = jnp.asarray(_SEL)
    exp = jnp.asarray(_EXP)
    freq8 = _pad8(params['freq'].reshape(-1, 1), 8).reshape(1, 8)

    x_col = (distances / R_CUTOFF).reshape(-1, 1)
    rbf8, rad48 = _basis(x_col, zflat, nflat, sel, freq8)

    cbf8 = _cbf(angles.reshape(-1, 1),
                angle_mask.astype(jnp.float32).reshape(-1, 1))

    expand_i32 = expand_to_kj.astype(jnp.int32)
    rad_g = _gather_rows(rad48, expand_i32)
    sbf = _sbf48(rad_g, cbf8, exp)

    species_p = jnp.zeros((N_ATOMS_PAD,), jnp.int32).at[:species.shape[0]].set(
        species.astype(jnp.int32))
    h = _gather_rows(params['emb'], species_p)
    hj = _gather_rows(h, idx_j.astype(jnp.int32))
    hi = _gather_rows(h, idx_i.astype(jnp.int32))

    we = params['W_edge']
    m, prod = _edge_embed(
        hj, hi, rbf8,
        _pad8(params['W_rbf_emb']), params['b_rbf_emb'].reshape(1, -1),
        we[:64], we[64:128], we[128:], params['b_edge'].reshape(1, -1),
        _pad8(params['out_blocks'][0]['W_rbf']))

    out_acc = jnp.zeros((N_ATOMS_PAD, 8), jnp.float32)
    idx_i32 = idx_i.astype(jnp.int32)
    reduce_i32 = reduce_to_ji.astype(jnp.int32)
    z_atoms = jnp.zeros((N_ATOMS_PAD // 16, 128), jnp.float32)
    z_tri = jnp.zeros(((_TRI_R + 16) // 16, 64), jnp.float32)
    seg_atoms = _make_sc_segsum_atoms(N_EDGES, 128, N_ATOMS_PAD)
    seg_tri = _make_sc_segsum_tri(N_ANGLES, 64, N_EDGES)

    for i in range(N_INTER + 1):
        pf = seg_atoms(prod, idx_i32, z_atoms)
        out_acc = _out_mlp(pf[:N_ATOMS_PAD], pf[N_ATOMS_PAD:], out_acc,
                           params['out_blocks'][i])
        if i == N_INTER:
            break
        ip = params['int_blocks'][i]
        wrbfc = _pad8(_mm(ip['W_rbf1'], ip['W_rbf2']))
        wc48 = _pad8(_mm(ip['W_sbf1'], ip['W_sbf2']), 48)
        xji, xdown = _inter_a(m, rbf8, ip['W_ji'], ip['b_ji'].reshape(1, -1),
                              ip['W_kj'], ip['b_kj'].reshape(1, -1),
                              wrbfc, ip['W_down'])
        gx = _gather_rows(xdown, expand_i32)
        t = _tri_t(gx, sbf, wc48)
        agg = seg_tri(t, reduce_i32, z_tri)
        m, prod = _inter_b(agg, xji, m, rbf8, ip['W_up'],
                           ip['res_before'][0], ip['W_skip'],
                           ip['b_skip'].reshape(1, -1),
                           ip['res_after'][0], ip['res_after'][1],
                           _pad8(params['out_blocks'][i + 1]['W_rbf']))

    return out_acc[:10000, :1]


def kernel(distances, angles, params, species, idx_i, idx_j, angle_mask,
           reduce_to_ji, expand_to_kj):
    return _forward(distances, angles, params, species, idx_i, idx_j,
                    angle_mask, reduce_to_ji, expand_to_kj)


# speedup vs baseline: 1.4880x; 1.0721x over previous
import functools

import jax
import jax.numpy as jnp
import numpy as np
from jax import lax
from jax.experimental import pallas as pl
from jax.experimental.pallas import tpu as pltpu
from jax.experimental.pallas import tpu_sc as plsc

R_CUTOFF = 5.0
NUM_RBF = 6
NUM_SBF = 7
EMBED = 128
ENV_P = 6
ANGLE_EMB = 64
OUT_EMB = 256
N_INTER = 4

N_EDGES = 160000
N_ANGLES = 320000
N_ATOMS_PAD = 10240

BE = 1000
BA = 1000
BP = 1024


def _sph_jl_np(l, x):
    x = np.asarray(x, dtype=np.float64)
    j0 = np.sin(x) / x
    if l == 0:
        return j0
    j1 = np.sin(x) / x**2 - np.cos(x) / x
    if l == 1:
        return j1
    jm, jc = j0, j1
    for i in range(1, l):
        jn = (2 * i + 1) / x * jc - jm
        jm, jc = jc, jn
    return jc


def _bessel_zeros(num_l, num_n):
    zeros = np.zeros((num_l, num_n))
    xs = np.linspace(1e-2, 80.0, 160001)
    for l in range(num_l):
        vals = _sph_jl_np(l, xs)
        s = np.sign(vals)
        idx = np.where(s[:-1] * s[1:] < 0)[0][:num_n]
        for n, i in enumerate(idx):
            a, b = xs[i], xs[i + 1]
            fa = _sph_jl_np(l, np.array([a]))[0]
            for _ in range(60):
                mid = 0.5 * (a + b)
                fm = _sph_jl_np(l, np.array([mid]))[0]
                if fa * fm <= 0:
                    b = mid
                else:
                    a, fa = mid, fm
            zeros[l, n] = 0.5 * (a + b)
    return zeros


_ZEROS_NP = _bessel_zeros(NUM_SBF, NUM_RBF)
_NORM_NP = np.zeros((NUM_SBF, NUM_RBF))
for _l in range(NUM_SBF):
    _NORM_NP[_l] = np.sqrt(2.0 / R_CUTOFF**3) / np.abs(_sph_jl_np(_l + 1, _ZEROS_NP[_l]))

_LEG_NP = np.sqrt((2 * np.arange(NUM_SBF) + 1) / (4 * np.pi)).astype(np.float32)

_ZFLAT = np.ones((1, 48), np.float32)
_ZFLAT[0, :42] = _ZEROS_NP.reshape(-1).astype(np.float32)
_NFLAT = np.zeros((1, 48), np.float32)
_NFLAT[0, :42] = _NORM_NP.reshape(-1).astype(np.float32)
_SEL = np.zeros((8, 48), np.float32)
for _l in range(NUM_SBF):
    _SEL[_l, _l * 6:(_l + 1) * 6] = 1.0
_EXP = np.zeros((8, 48), np.float32)
for _l in range(NUM_SBF):
    _EXP[_l, _l * 6:(_l + 1) * 6] = 1.0


def _swish(x):
    return x / (1.0 + jnp.exp(-x))


def _mm(a, b):
    return jnp.dot(a, b, preferred_element_type=jnp.float32)


def _wspec(shape):
    nd = len(shape)
    return pl.BlockSpec(shape, lambda i, _n=nd: (0,) * _n)


def _rspec(shape):
    return pl.BlockSpec(shape, lambda i: (i,) + (0,) * (len(shape) - 1))


def _basis_body(x_ref, zflat_ref, nflat_ref, sel_ref, freq_ref, rbf_ref, rad_ref):
    x = x_ref[...]
    arg = x * zflat_ref[...]
    inv = 1.0 / x
    x2 = x * x
    x3 = x2 * x
    x6 = x3 * x3
    x7 = x6 * x
    x8 = x6 * x2
    p = ENV_P + 1
    a = -(p + 1) * (p + 2) / 2.0
    b = p * (p + 2)
    c = -p * (p + 1) / 2.0
    env = jnp.where(x < 1.0, inv + a * x6 + b * x7 + c * x8, 0.0)

    s = jnp.sin(arg)
    co = jnp.cos(arg)
    j0 = s / arg
    j1 = s / (arg * arg) - co / arg
    js = [j0, j1]
    for i in range(1, NUM_SBF - 1):
        js.append((2 * i + 1) / arg * js[i] - js[i - 1])
    sel = jnp.zeros_like(arg)
    for l in range(NUM_SBF):
        sel = sel + js[l] * sel_ref[l:l + 1, :]
    rad_ref[...] = sel * nflat_ref[...] * env
    rbf_ref[...] = env * jnp.sin(freq_ref[...] * x)


def _basis(x_col, zflat, nflat, sel, freq8):
    n = x_col.shape[0]
    return pl.pallas_call(
        _basis_body,
        grid=(n // BE,),
        in_specs=[_rspec((BE, 1)), _wspec((1, 48)), _wspec((1, 48)),
                  _wspec((8, 48)), _wspec((1, 8))],
        out_specs=[_rspec((BE, 8)), _rspec((BE, 48))],
        out_shape=[jax.ShapeDtypeStruct((n, 8), jnp.float32),
                   jax.ShapeDtypeStruct((n, 48), jnp.float32)],
    )(x_col, zflat, nflat, sel, freq8)


def _cbf_body(ang_ref, mask_ref, out_ref):
    ct = jnp.cos(ang_ref[...])
    ps = [jnp.ones_like(ct), ct]
    for l in range(1, NUM_SBF - 1):
        ps.append(((2 * l + 1) * ct * ps[l] - l * ps[l - 1]) / (l + 1))
    msk = mask_ref[...]
    cols = [_LEG_NP[l] * ps[l] * msk for l in range(NUM_SBF)]
    cols.append(jnp.zeros_like(ct))
    out_ref[...] = jnp.concatenate(cols, axis=1)


def _cbf(ang_col, mask_col):
    n = ang_col.shape[0]
    return pl.pallas_call(
        _cbf_body,
        grid=(n // BA,),
        in_specs=[_rspec((BA, 1)), _rspec((BA, 1))],
        out_specs=_rspec((BA, 8)),
        out_shape=jax.ShapeDtypeStruct((n, 8), jnp.float32),
    )(ang_col, mask_col)


def _sbf48_body(radg_ref, cbf_ref, exp_ref, out_ref):
    out_ref[...] = radg_ref[...] * _mm(cbf_ref[...], exp_ref[...])


def _sbf48(rad_g, cbf8, exp):
    n = rad_g.shape[0]
    return pl.pallas_call(
        _sbf48_body,
        grid=(n // BA,),
        in_specs=[_rspec((BA, 48)), _rspec((BA, 8)), _wspec((8, 48))],
        out_specs=_rspec((BA, 48)),
        out_shape=jax.ShapeDtypeStruct((n, 48), jnp.float32),
    )(rad_g, cbf8, exp)


def _edge_embed_body(hj_ref, hi_ref, rbf_ref, wre_ref, bre_ref, wj_ref,
                     wi_ref, wr_ref, be_ref, wrbf0_ref, m_ref, prod_ref):
    rbf = rbf_ref[...]
    rbf_e = _swish(_mm(rbf, wre_ref[...]) + bre_ref[...])
    m = _swish(_mm(hj_ref[...], wj_ref[...]) + _mm(hi_ref[...], wi_ref[...])
               + _mm(rbf_e, wr_ref[...]) + be_ref[...])
    m_ref[...] = m
    prod_ref[...] = _mm(rbf, wrbf0_ref[...]) * m


def _edge_embed(hj, hi, rbf8, wre, bre, wj, wi, wr, be_, wrbf0):
    n = hj.shape[0]
    return pl.pallas_call(
        _edge_embed_body,
        grid=(n // BE,),
        in_specs=[_rspec((BE, 64)), _rspec((BE, 64)), _rspec((BE, 8)),
                  _wspec((8, 128)), _wspec((1, 128)), _wspec((64, 128)),
                  _wspec((64, 128)), _wspec((128, 128)), _wspec((1, 128)),
                  _wspec((8, 128))],
        out_specs=[_rspec((BE, 128)), _rspec((BE, 128))],
        out_shape=[jax.ShapeDtypeStruct((n, 128), jnp.float32),
                   jax.ShapeDtypeStruct((n, 128), jnp.float32)],
    )(hj, hi, rbf8, wre, bre, wj, wi, wr, be_, wrbf0)


def _inter_a_body(m_ref, rbf_ref, wji_ref, bji_ref, wkj_ref, bkj_ref,
                  wrbfc_ref, wdown_ref, xji_ref, xdown_ref):
    m = m_ref[...]
    xji_ref[...] = _swish(_mm(m, wji_ref[...]) + bji_ref[...])
    x_kj = _swish(_mm(m, wkj_ref[...]) + bkj_ref[...])
    x_kj = x_kj * _mm(rbf_ref[...], wrbfc_ref[...])
    xdown_ref[...] = _swish(_mm(x_kj, wdown_ref[...]))


def _inter_a(m, rbf8, wji, bji, wkj, bkj, wrbfc, wdown):
    n = m.shape[0]
    return pl.pallas_call(
        _inter_a_body,
        grid=(n // BE,),
        in_specs=[_rspec((BE, 128)), _rspec((BE, 8)), _wspec((128, 128)),
                  _wspec((1, 128)), _wspec((128, 128)), _wspec((1, 128)),
                  _wspec((8, 128)), _wspec((128, 64))],
        out_specs=[_rspec((BE, 128)), _rspec((BE, 64))],
        out_shape=[jax.ShapeDtypeStruct((n, 128), jnp.float32),
                   jax.ShapeDtypeStruct((n, 64), jnp.float32)],
    )(m, rbf8, wji, bji, wkj, bkj, wrbfc, wdown)


def _tri_t_body(gx_ref, sbf_ref, wc_ref, out_ref):
    out_ref[...] = gx_ref[...] * _mm(sbf_ref[...], wc_ref[...])


def _tri_t(gx, sbf48, wc):
    n = gx.shape[0]
    return pl.pallas_call(
        _tri_t_body,
        grid=(n // BA,),
        in_specs=[_rspec((BA, 64)), _rspec((BA, 48)), _wspec((48, 64))],
        out_specs=_rspec((BA, 64)),
        out_shape=jax.ShapeDtypeStruct((n, 64), jnp.float32),
    )(gx, sbf48, wc)


def _inter_b_body(agg_ref, xji_ref, m_ref, rbf_ref, wup_ref,
                  rb_w1, rb_b1, rb_w2, rb_b2, wskip_ref, bskip_ref,
                  ra1_w1, ra1_b1, ra1_w2, ra1_b2,
                  ra2_w1, ra2_b1, ra2_w2, ra2_b2, wrbfn_ref,
                  mnew_ref, prod_ref):
    x_kj = _swish(_mm(agg_ref[...], wup_ref[...]))
    h = xji_ref[...] + x_kj
    h = h + _swish(_mm(_swish(_mm(h, rb_w1[...]) + rb_b1[...]), rb_w2[...]) + rb_b2[...])
    h = _swish(_mm(h, wskip_ref[...]) + bskip_ref[...]) + m_ref[...]
    h = h + _swish(_mm(_swish(_mm(h, ra1_w1[...]) + ra1_b1[...]), ra1_w2[...]) + ra1_b2[...])
    h = h + _swish(_mm(_swish(_mm(h, ra2_w1[...]) + ra2_b1[...]), ra2_w2[...]) + ra2_b2[...])
    mnew_ref[...] = h
    prod_ref[...] = _mm(rbf_ref[...], wrbfn_ref[...]) * h


def _inter_b(agg, xji, m, rbf8, wup, rb, wskip, bskip, ra1, ra2, wrbfn):
    n = m.shape[0]
    w128 = _wspec((128, 128))
    b128 = _wspec((1, 128))
    return pl.pallas_call(
        _inter_b_body,
        grid=(n // BE,),
        in_specs=[_rspec((BE, 64)), _rspec((BE, 128)), _rspec((BE, 128)),
                  _rspec((BE, 8)), _wspec((64, 128)),
                  w128, b128, w128, b128, w128, b128,
                  w128, b128, w128, b128,
                  w128, b128, w128, b128, _wspec((8, 128))],
        out_specs=[_rspec((BE, 128)), _rspec((BE, 128))],
        out_shape=[jax.ShapeDtypeStruct((n, 128), jnp.float32),
                   jax.ShapeDtypeStruct((n, 128), jnp.float32)],
    )(agg, xji, m, rbf8, wup,
      rb['W1'], rb['b1'].reshape(1, -1), rb['W2'], rb['b2'].reshape(1, -1),
      wskip, bskip,
      ra1['W1'], ra1['b1'].reshape(1, -1), ra1['W2'], ra1['b2'].reshape(1, -1),
      ra2['W1'], ra2['b1'].reshape(1, -1), ra2['W2'], ra2['b2'].reshape(1, -1),
      wrbfn)


def _out_mlp_body(p0_ref, p1_ref, prev_ref, wup_ref, w1, b1, w2, b2, w3, b3,
                  wout_ref, out_ref):
    t = _mm(p0_ref[...] + p1_ref[...], wup_ref[...])
    t = _swish(_mm(t, w1[...]) + b1[...])
    t = _swish(_mm(t, w2[...]) + b2[...])
    t = _swish(_mm(t, w3[...]) + b3[...])
    out_ref[...] = prev_ref[...] + _mm(t, wout_ref[...])


def _out_mlp(p0, p1, prev, ob):
    n = p0.shape[0]
    wout8 = jnp.zeros((OUT_EMB, 8), jnp.float32).at[:, :1].set(ob['W_out'])
    w256 = _wspec((256, 256))
    b256 = _wspec((1, 256))
    return pl.pallas_call(
        _out_mlp_body,
        grid=(n // BP,),
        in_specs=[_rspec((BP, 128)), _rspec((BP, 128)), _rspec((BP, 8)),
                  _wspec((128, 256)), w256, b256, w256, b256, w256, b256,
                  _wspec((256, 8))],
        out_specs=_rspec((BP, 8)),
        out_shape=jax.ShapeDtypeStruct((n, 8), jnp.float32),
    )(p0, p1, prev, ob['W_up'],
      ob['Ws'][0], ob['bs'][0].reshape(1, -1),
      ob['Ws'][1], ob['bs'][1].reshape(1, -1),
      ob['Ws'][2], ob['bs'][2].reshape(1, -1),
      wout8)


_NW = 32


@functools.lru_cache(maxsize=None)
def _make_sc_gather(V, D, B):
    assert D % 16 == 0 and B % (8 * _NW) == 0
    b_per_w = B // _NW
    nb, rem = divmod(b_per_w, 128)
    mesh = plsc.VectorSubcoreMesh(core_axis_name="c", subcore_axis_name="s")

    @functools.partial(
        pl.kernel, mesh=mesh,
        out_type=jax.ShapeDtypeStruct((B, D), jnp.float32),
        compiler_params=pltpu.CompilerParams(use_tc_tiling_on_sc=False),
        scratch_types=[
            pltpu.VMEM((b_per_w,), jnp.int32),
            pltpu.VMEM((128, D), jnp.float32),
            pltpu.SemaphoreType.DMA,
        ],
    )
    def k(table_hbm, idx_hbm, out_hbm, idx_v, rows_v, sem):
        wid = lax.axis_index("s") * 2 + lax.axis_index("c")
        base = wid * b_per_w
        pltpu.sync_copy(idx_hbm.at[pl.ds(base, b_per_w)], idx_v)

        def body(j, _):
            off = j * 128
            pltpu.async_copy(
                table_hbm.at[idx_v.at[pl.ds(off, 128)]], rows_v, sem).wait()
            pltpu.sync_copy(rows_v, out_hbm.at[pl.ds(base + off, 128)])
            return _

        lax.fori_loop(0, nb, body, 0)
        if rem:
            off = nb * 128
            pltpu.async_copy(
                table_hbm.at[idx_v.at[pl.ds(off, rem)]],
                rows_v.at[pl.ds(0, rem)], sem).wait()
            pltpu.sync_copy(rows_v.at[pl.ds(0, rem)],
                            out_hbm.at[pl.ds(base + off, rem)])

    return k


def _gather_rows(table, idx):
    V, D = table.shape
    B = idx.shape[0]
    return _make_sc_gather(V, D, B)(table, idx)


def _segsum(rows, idx, nseg):
    return jax.ops.segment_sum(rows, idx, num_segments=nseg)


@functools.lru_cache(maxsize=None)
def _make_sc_segsum_atoms(E, D, NSEG):
    assert E % _NW == 0 and NSEG % 16 == 0
    e_per_w = E // _NW
    nb, rem = divmod(e_per_w, 128)
    stripe = NSEG // 16
    mesh = plsc.VectorSubcoreMesh(core_axis_name="c", subcore_axis_name="s")

    @functools.partial(
        pl.kernel, mesh=mesh,
        out_type=jax.ShapeDtypeStruct((2 * NSEG, D), jnp.float32),
        compiler_params=pltpu.CompilerParams(use_tc_tiling_on_sc=False),
        scratch_types=[
            pltpu.VMEM_SHARED((NSEG, D), jnp.float32),
            pltpu.VMEM((128,), jnp.int32),
            pltpu.VMEM((8,), jnp.int32),
            pltpu.VMEM((128, D), jnp.float32),
            pltpu.SemaphoreType.DMA,
        ],
    )
    def k(src_hbm, idx_hbm, zero_hbm, out_hbm, shared, idx_v, idxt_v, rows_v, sem):
        c = lax.axis_index("c")
        s = lax.axis_index("s")
        wid = s * 2 + c
        base = wid * e_per_w
        pltpu.sync_copy(zero_hbm, shared.at[pl.ds(s * stripe, stripe)])
        plsc.subcore_barrier()

        def body(j, _):
            off = base + j * 128
            pltpu.sync_copy(idx_hbm.at[pl.ds(off, 128)], idx_v)
            pltpu.sync_copy(src_hbm.at[pl.ds(off, 128)], rows_v)
            pltpu.sync_copy(rows_v, shared.at[idx_v], add=True)
            return _

        lax.fori_loop(0, nb, body, 0)
        if rem:
            off = base + nb * 128
            pltpu.sync_copy(idx_hbm.at[pl.ds(off, rem)], idxt_v)
            pltpu.sync_copy(src_hbm.at[pl.ds(off, rem)], rows_v.at[pl.ds(0, rem)])
            pltpu.sync_copy(rows_v.at[pl.ds(0, rem)], shared.at[idxt_v], add=True)
        plsc.subcore_barrier()
        pltpu.sync_copy(shared.at[pl.ds(s * stripe, stripe)],
                        out_hbm.at[pl.ds(c * NSEG + s * stripe, stripe)])

    return k


_TRI_R = 40000
_TRI_NRANGE = 4


_SB = 512
_NSUB = _SB // 128


@functools.lru_cache(maxsize=None)
def _make_sc_segsum_tri(A, D, NSEG):
    DH = D // 2
    assert A % 16 == 0 and NSEG * 2 == _TRI_R * _TRI_NRANGE * 2
    a_per_t = A // 16
    nb, rem = divmod(a_per_t, _SB)
    assert rem % 16 == 0 and rem <= 128
    zstripe = (_TRI_R + 16) // 16
    ostripe = _TRI_R // 16
    mesh = plsc.VectorSubcoreMesh(core_axis_name="c", subcore_axis_name="s")

    loc_scratch = [pltpu.VMEM((128,), jnp.int32) for _ in range(_NSUB)]

    @functools.partial(
        pl.kernel, mesh=mesh,
        out_type=jax.ShapeDtypeStruct((NSEG, D), jnp.float32),
        compiler_params=pltpu.CompilerParams(use_tc_tiling_on_sc=False),
        scratch_types=[
            pltpu.VMEM_SHARED((_TRI_R + 16, DH), jnp.float32),
            pltpu.VMEM((_SB,), jnp.int32),
            pltpu.VMEM((32,), jnp.int32),
            pltpu.VMEM((_SB, DH), jnp.float32),
        ] + loc_scratch,
    )
    def k(src_hbm, idx_hbm, zero_hbm, out_hbm, shared, idx_v, loct_v, rows_v,
          *locs):
        c = lax.axis_index("c")
        s = lax.axis_index("s")
        tbase = s * a_per_t

        dump = _TRI_R + s

        def localize(n, dst, voff, rng_base):
            for v in range(n // 16):
                iv = idx_v[pl.ds(voff * 128 + v * 16, 16)]
                loc = iv - rng_base
                oob = (loc < 0) | (loc >= _TRI_R)
                dst[pl.ds(v * 16, 16)] = jnp.where(oob, dump, loc)

        for half in range(2):
            cs = half * DH
            for pr in range(_TRI_NRANGE // 2):
                rng_base = (c * (_TRI_NRANGE // 2) + pr) * _TRI_R
                pltpu.sync_copy(zero_hbm,
                                shared.at[pl.ds(s * zstripe, zstripe)])
                plsc.subcore_barrier()

                def body(g, carry):
                    off = tbase + g * _SB
                    pltpu.sync_copy(idx_hbm.at[pl.ds(off, _SB)], idx_v)
                    pltpu.sync_copy(
                        src_hbm.at[pl.ds(off, _SB), pl.ds(cs, DH)], rows_v)
                    for q in range(_NSUB):
                        localize(128, locs[q], q, rng_base)
                        pltpu.sync_copy(rows_v.at[pl.ds(q * 128, 128)],
                                        shared.at[locs[q]], add=True)
                    return carry

                lax.fori_loop(0, nb, body, 0)
                if rem:
                    off = tbase + nb * _SB
                    pltpu.sync_copy(idx_hbm.at[pl.ds(off, rem)],
                                    idx_v.at[pl.ds(0, rem)])
                    localize(rem, loct_v, 0, rng_base)
                    pltpu.sync_copy(
                        src_hbm.at[pl.ds(off, rem), pl.ds(cs, DH)],
                        rows_v.at[pl.ds(0, rem)])
                    pltpu.sync_copy(rows_v.at[pl.ds(0, rem)],
                                    shared.at[loct_v], add=True)
                plsc.subcore_barrier()
                pltpu.sync_copy(
                    shared.at[pl.ds(s * ostripe, ostripe)],
                    out_hbm.at[pl.ds(rng_base + s * ostripe, ostripe),
                               pl.ds(cs, DH)])
                plsc.subcore_barrier()

    return k


def _pad8(w, rows=8):
    out = jnp.zeros((rows,) + w.shape[1:], w.dtype)
    return out.at[:w.shape[0]].set(w)


@jax.jit
def _forward(distances, angles, params, species, idx_i, idx_j, angle_mask,
             reduce_to_ji, expand_to_kj):
    zflat = jnp.asarray(_ZFLAT)
    nflat = jnp.asarray(_NFLAT)
    sel = jnp.asarray(_SEL)
    exp = jnp.asarray(_EXP)
    freq8 = _pad8(params['freq'].reshape(-1, 1), 8).reshape(1, 8)

    x_col = (distances / R_CUTOFF).reshape(-1, 1)
    rbf8, rad48 = _basis(x_col, zflat, nflat, sel, freq8)

    cbf8 = _cbf(angles.reshape(-1, 1),
                angle_mask.astype(jnp.float32).reshape(-1, 1))

    expand_i32 = expand_to_kj.astype(jnp.int32)
    rad_g = _gather_rows(rad48, expand_i32)
    sbf = _sbf48(rad_g, cbf8, exp)

    species_p = jnp.zeros((N_ATOMS_PAD,), jnp.int32).at[:species.shape[0]].set(
        species.astype(jnp.int32))
    h = _gather_rows(params['emb'], species_p)
    hj = _gather_rows(h, idx_j.astype(jnp.int32))
    hi = _gather_rows(h, idx_i.astype(jnp.int32))

    we = params['W_edge']
    m, prod = _edge_embed(
        hj, hi, rbf8,
        _pad8(params['W_rbf_emb']), params['b_rbf_emb'].reshape(1, -1),
        we[:64], we[64:128], we[128:], params['b_edge'].reshape(1, -1),
        _pad8(params['out_blocks'][0]['W_rbf']))

    out_acc = jnp.zeros((N_ATOMS_PAD, 8), jnp.float32)
    idx_i32 = idx_i.astype(jnp.int32)
    reduce_i32 = reduce_to_ji.astype(jnp.int32)
    z_atoms = jnp.zeros((N_ATOMS_PAD // 16, 128), jnp.float32)
    z_tri = jnp.zeros(((_TRI_R + 16) // 16, 32), jnp.float32)
    seg_atoms = _make_sc_segsum_atoms(N_EDGES, 128, N_ATOMS_PAD)
    seg_tri = _make_sc_segsum_tri(N_ANGLES, 64, N_EDGES)

    for i in range(N_INTER + 1):
        pf = seg_atoms(prod, idx_i32, z_atoms)
        out_acc = _out_mlp(pf[:N_ATOMS_PAD], pf[N_ATOMS_PAD:], out_acc,
                           params['out_blocks'][i])
        if i == N_INTER:
            break
        ip = params['int_blocks'][i]
        wrbfc = _pad8(_mm(ip['W_rbf1'], ip['W_rbf2']))
        wc48 = _pad8(_mm(ip['W_sbf1'], ip['W_sbf2']), 48)
        xji, xdown = _inter_a(m, rbf8, ip['W_ji'], ip['b_ji'].reshape(1, -1),
                              ip['W_kj'], ip['b_kj'].reshape(1, -1),
                              wrbfc, ip['W_down'])
        gx = _gather_rows(xdown, expand_i32)
        t = _tri_t(gx, sbf, wc48)
        agg = seg_tri(t, reduce_i32, z_tri)
        m, prod = _inter_b(agg, xji, m, rbf8, ip['W_up'],
                           ip['res_before'][0], ip['W_skip'],
                           ip['b_skip'].reshape(1, -1),
                           ip['res_after'][0], ip['res_after'][1],
                           _pad8(params['out_blocks'][i + 1]['W_rbf']))

    return out_acc[:10000, :1]


def kernel(distances, angles, params, species, idx_i, idx_j, angle_mask,
           reduce_to_ji, expand_to_kj):
    return _forward(distances, angles, params, species, idx_i, idx_j,
                    angle_mask, reduce_to_ji, expand_to_kj)


# speedup vs baseline: 1.4903x; 1.0015x over previous
import functools

import jax
import jax.numpy as jnp
import numpy as np
from jax import lax
from jax.experimental import pallas as pl
from jax.experimental.pallas import tpu as pltpu
from jax.experimental.pallas import tpu_sc as plsc

R_CUTOFF = 5.0
NUM_RBF = 6
NUM_SBF = 7
EMBED = 128
ENV_P = 6
ANGLE_EMB = 64
OUT_EMB = 256
N_INTER = 4

N_EDGES = 160000
N_ANGLES = 320000
N_ATOMS_PAD = 10240

BE = 1000
BA = 1000
BP = 1024


def _sph_jl_np(l, x):
    x = np.asarray(x, dtype=np.float64)
    j0 = np.sin(x) / x
    if l == 0:
        return j0
    j1 = np.sin(x) / x**2 - np.cos(x) / x
    if l == 1:
        return j1
    jm, jc = j0, j1
    for i in range(1, l):
        jn = (2 * i + 1) / x * jc - jm
        jm, jc = jc, jn
    return jc


def _bessel_zeros(num_l, num_n):
    zeros = np.zeros((num_l, num_n))
    xs = np.linspace(1e-2, 80.0, 160001)
    for l in range(num_l):
        vals = _sph_jl_np(l, xs)
        s = np.sign(vals)
        idx = np.where(s[:-1] * s[1:] < 0)[0][:num_n]
        for n, i in enumerate(idx):
            a, b = xs[i], xs[i + 1]
            fa = _sph_jl_np(l, np.array([a]))[0]
            for _ in range(60):
                mid = 0.5 * (a + b)
                fm = _sph_jl_np(l, np.array([mid]))[0]
                if fa * fm <= 0:
                    b = mid
                else:
                    a, fa = mid, fm
            zeros[l, n] = 0.5 * (a + b)
    return zeros


_ZEROS_NP = _bessel_zeros(NUM_SBF, NUM_RBF)
_NORM_NP = np.zeros((NUM_SBF, NUM_RBF))
for _l in range(NUM_SBF):
    _NORM_NP[_l] = np.sqrt(2.0 / R_CUTOFF**3) / np.abs(_sph_jl_np(_l + 1, _ZEROS_NP[_l]))

_LEG_NP = np.sqrt((2 * np.arange(NUM_SBF) + 1) / (4 * np.pi)).astype(np.float32)

_ZFLAT = np.ones((1, 48), np.float32)
_ZFLAT[0, :42] = _ZEROS_NP.reshape(-1).astype(np.float32)
_NFLAT = np.zeros((1, 48), np.float32)
_NFLAT[0, :42] = _NORM_NP.reshape(-1).astype(np.float32)
_SEL = np.zeros((8, 48), np.float32)
for _l in range(NUM_SBF):
    _SEL[_l, _l * 6:(_l + 1) * 6] = 1.0
_EXP = np.zeros((8, 48), np.float32)
for _l in range(NUM_SBF):
    _EXP[_l, _l * 6:(_l + 1) * 6] = 1.0


def _swish(x):
    return x / (1.0 + jnp.exp(-x))


def _mm(a, b):
    return jnp.dot(a, b, preferred_element_type=jnp.float32)


def _wspec(shape):
    nd = len(shape)
    return pl.BlockSpec(shape, lambda i, _n=nd: (0,) * _n)


def _rspec(shape):
    return pl.BlockSpec(shape, lambda i: (i,) + (0,) * (len(shape) - 1))


def _basis_body(x_ref, zflat_ref, nflat_ref, sel_ref, freq_ref, rbf_ref, rad_ref):
    x = x_ref[...]
    arg = x * zflat_ref[...]
    inv = 1.0 / x
    x2 = x * x
    x3 = x2 * x
    x6 = x3 * x3
    x7 = x6 * x
    x8 = x6 * x2
    p = ENV_P + 1
    a = -(p + 1) * (p + 2) / 2.0
    b = p * (p + 2)
    c = -p * (p + 1) / 2.0
    env = jnp.where(x < 1.0, inv + a * x6 + b * x7 + c * x8, 0.0)

    s = jnp.sin(arg)
    co = jnp.cos(arg)
    j0 = s / arg
    j1 = s / (arg * arg) - co / arg
    js = [j0, j1]
    for i in range(1, NUM_SBF - 1):
        js.append((2 * i + 1) / arg * js[i] - js[i - 1])
    sel = jnp.zeros_like(arg)
    for l in range(NUM_SBF):
        sel = sel + js[l] * sel_ref[l:l + 1, :]
    rad_ref[...] = sel * nflat_ref[...] * env
    rbf_ref[...] = env * jnp.sin(freq_ref[...] * x)


def _basis(x_col, zflat, nflat, sel, freq8):
    n = x_col.shape[0]
    return pl.pallas_call(
        _basis_body,
        grid=(n // BE,),
        in_specs=[_rspec((BE, 1)), _wspec((1, 48)), _wspec((1, 48)),
                  _wspec((8, 48)), _wspec((1, 8))],
        out_specs=[_rspec((BE, 8)), _rspec((BE, 48))],
        out_shape=[jax.ShapeDtypeStruct((n, 8), jnp.float32),
                   jax.ShapeDtypeStruct((n, 48), jnp.float32)],
    )(x_col, zflat, nflat, sel, freq8)


def _cbf_body(ang_ref, mask_ref, out_ref):
    ct = jnp.cos(ang_ref[...])
    ps = [jnp.ones_like(ct), ct]
    for l in range(1, NUM_SBF - 1):
        ps.append(((2 * l + 1) * ct * ps[l] - l * ps[l - 1]) / (l + 1))
    msk = mask_ref[...]
    cols = [_LEG_NP[l] * ps[l] * msk for l in range(NUM_SBF)]
    cols.append(jnp.zeros_like(ct))
    out_ref[...] = jnp.concatenate(cols, axis=1)


def _cbf(ang_col, mask_col):
    n = ang_col.shape[0]
    return pl.pallas_call(
        _cbf_body,
        grid=(n // BA,),
        in_specs=[_rspec((BA, 1)), _rspec((BA, 1))],
        out_specs=_rspec((BA, 8)),
        out_shape=jax.ShapeDtypeStruct((n, 8), jnp.float32),
    )(ang_col, mask_col)


def _sbf48_body(radg_ref, cbf_ref, exp_ref, out_ref):
    out_ref[...] = radg_ref[...] * _mm(cbf_ref[...], exp_ref[...])


def _sbf48(rad_g, cbf8, exp):
    n = rad_g.shape[0]
    return pl.pallas_call(
        _sbf48_body,
        grid=(n // BA,),
        in_specs=[_rspec((BA, 48)), _rspec((BA, 8)), _wspec((8, 48))],
        out_specs=_rspec((BA, 48)),
        out_shape=jax.ShapeDtypeStruct((n, 48), jnp.float32),
    )(rad_g, cbf8, exp)


def _edge_embed_body(hj_ref, hi_ref, rbf_ref, wre_ref, bre_ref, wj_ref,
                     wi_ref, wr_ref, be_ref, wrbf0_ref, m_ref, prod_ref):
    rbf = rbf_ref[...]
    rbf_e = _swish(_mm(rbf, wre_ref[...]) + bre_ref[...])
    m = _swish(_mm(hj_ref[...], wj_ref[...]) + _mm(hi_ref[...], wi_ref[...])
               + _mm(rbf_e, wr_ref[...]) + be_ref[...])
    m_ref[...] = m
    prod_ref[...] = _mm(rbf, wrbf0_ref[...]) * m


def _edge_embed(hj, hi, rbf8, wre, bre, wj, wi, wr, be_, wrbf0):
    n = hj.shape[0]
    return pl.pallas_call(
        _edge_embed_body,
        grid=(n // BE,),
        in_specs=[_rspec((BE, 64)), _rspec((BE, 64)), _rspec((BE, 8)),
                  _wspec((8, 128)), _wspec((1, 128)), _wspec((64, 128)),
                  _wspec((64, 128)), _wspec((128, 128)), _wspec((1, 128)),
                  _wspec((8, 128))],
        out_specs=[_rspec((BE, 128)), _rspec((BE, 128))],
        out_shape=[jax.ShapeDtypeStruct((n, 128), jnp.float32),
                   jax.ShapeDtypeStruct((n, 128), jnp.float32)],
    )(hj, hi, rbf8, wre, bre, wj, wi, wr, be_, wrbf0)


def _inter_a_body(m_ref, rbf_ref, wji_ref, bji_ref, wkj_ref, bkj_ref,
                  wrbfc_ref, wdown_ref, xji_ref, xdown_ref):
    m = m_ref[...]
    xji_ref[...] = _swish(_mm(m, wji_ref[...]) + bji_ref[...])
    x_kj = _swish(_mm(m, wkj_ref[...]) + bkj_ref[...])
    x_kj = x_kj * _mm(rbf_ref[...], wrbfc_ref[...])
    xdown_ref[...] = _swish(_mm(x_kj, wdown_ref[...]))


def _inter_a(m, rbf8, wji, bji, wkj, bkj, wrbfc, wdown):
    n = m.shape[0]
    return pl.pallas_call(
        _inter_a_body,
        grid=(n // BE,),
        in_specs=[_rspec((BE, 128)), _rspec((BE, 8)), _wspec((128, 128)),
                  _wspec((1, 128)), _wspec((128, 128)), _wspec((1, 128)),
                  _wspec((8, 128)), _wspec((128, 64))],
        out_specs=[_rspec((BE, 128)), _rspec((BE, 64))],
        out_shape=[jax.ShapeDtypeStruct((n, 128), jnp.float32),
                   jax.ShapeDtypeStruct((n, 64), jnp.float32)],
    )(m, rbf8, wji, bji, wkj, bkj, wrbfc, wdown)


def _tri_t_body(gx_ref, sbf_ref, wc_ref, out_ref):
    out_ref[...] = gx_ref[...] * _mm(sbf_ref[...], wc_ref[...])


def _tri_t(gx, sbf48, wc):
    n = gx.shape[0]
    return pl.pallas_call(
        _tri_t_body,
        grid=(n // BA,),
        in_specs=[_rspec((BA, 64)), _rspec((BA, 48)), _wspec((48, 64))],
        out_specs=_rspec((BA, 64)),
        out_shape=jax.ShapeDtypeStruct((n, 64), jnp.float32),
    )(gx, sbf48, wc)


def _inter_b_body(agg_ref, xji_ref, m_ref, rbf_ref, wup_ref,
                  rb_w1, rb_b1, rb_w2, rb_b2, wskip_ref, bskip_ref,
                  ra1_w1, ra1_b1, ra1_w2, ra1_b2,
                  ra2_w1, ra2_b1, ra2_w2, ra2_b2, wrbfn_ref,
                  mnew_ref, prod_ref):
    x_kj = _swish(_mm(agg_ref[...], wup_ref[...]))
    h = xji_ref[...] + x_kj
    h = h + _swish(_mm(_swish(_mm(h, rb_w1[...]) + rb_b1[...]), rb_w2[...]) + rb_b2[...])
    h = _swish(_mm(h, wskip_ref[...]) + bskip_ref[...]) + m_ref[...]
    h = h + _swish(_mm(_swish(_mm(h, ra1_w1[...]) + ra1_b1[...]), ra1_w2[...]) + ra1_b2[...])
    h = h + _swish(_mm(_swish(_mm(h, ra2_w1[...]) + ra2_b1[...]), ra2_w2[...]) + ra2_b2[...])
    mnew_ref[...] = h
    prod_ref[...] = _mm(rbf_ref[...], wrbfn_ref[...]) * h


def _inter_b(agg, xji, m, rbf8, wup, rb, wskip, bskip, ra1, ra2, wrbfn):
    n = m.shape[0]
    w128 = _wspec((128, 128))
    b128 = _wspec((1, 128))
    return pl.pallas_call(
        _inter_b_body,
        grid=(n // BE,),
        in_specs=[_rspec((BE, 64)), _rspec((BE, 128)), _rspec((BE, 128)),
                  _rspec((BE, 8)), _wspec((64, 128)),
                  w128, b128, w128, b128, w128, b128,
                  w128, b128, w128, b128,
                  w128, b128, w128, b128, _wspec((8, 128))],
        out_specs=[_rspec((BE, 128)), _rspec((BE, 128))],
        out_shape=[jax.ShapeDtypeStruct((n, 128), jnp.float32),
                   jax.ShapeDtypeStruct((n, 128), jnp.float32)],
    )(agg, xji, m, rbf8, wup,
      rb['W1'], rb['b1'].reshape(1, -1), rb['W2'], rb['b2'].reshape(1, -1),
      wskip, bskip,
      ra1['W1'], ra1['b1'].reshape(1, -1), ra1['W2'], ra1['b2'].reshape(1, -1),
      ra2['W1'], ra2['b1'].reshape(1, -1), ra2['W2'], ra2['b2'].reshape(1, -1),
      wrbfn)


def _out_mlp_body(p0_ref, p1_ref, prev_ref, wup_ref, w1, b1, w2, b2, w3, b3,
                  wout_ref, out_ref):
    t = _mm(p0_ref[...] + p1_ref[...], wup_ref[...])
    t = _swish(_mm(t, w1[...]) + b1[...])
    t = _swish(_mm(t, w2[...]) + b2[...])
    t = _swish(_mm(t, w3[...]) + b3[...])
    out_ref[...] = prev_ref[...] + _mm(t, wout_ref[...])


def _out_mlp(p0, p1, prev, ob):
    n = p0.shape[0]
    wout8 = jnp.zeros((OUT_EMB, 8), jnp.float32).at[:, :1].set(ob['W_out'])
    w256 = _wspec((256, 256))
    b256 = _wspec((1, 256))
    return pl.pallas_call(
        _out_mlp_body,
        grid=(n // BP,),
        in_specs=[_rspec((BP, 128)), _rspec((BP, 128)), _rspec((BP, 8)),
                  _wspec((128, 256)), w256, b256, w256, b256, w256, b256,
                  _wspec((256, 8))],
        out_specs=_rspec((BP, 8)),
        out_shape=jax.ShapeDtypeStruct((n, 8), jnp.float32),
    )(p0, p1, prev, ob['W_up'],
      ob['Ws'][0], ob['bs'][0].reshape(1, -1),
      ob['Ws'][1], ob['bs'][1].reshape(1, -1),
      ob['Ws'][2], ob['bs'][2].reshape(1, -1),
      wout8)


_NW = 32


@functools.lru_cache(maxsize=None)
def _make_sc_gather(V, D, B):
    assert D % 16 == 0 and B % (8 * _NW) == 0
    b_per_w = B // _NW
    nb, rem = divmod(b_per_w, 128)
    mesh = plsc.VectorSubcoreMesh(core_axis_name="c", subcore_axis_name="s")

    chunks = []
    off = 0
    while off < b_per_w:
        csz = min(512, b_per_w - off)
        subs = []
        so = 0
        while so < csz:
            ssz = min(128, csz - so)
            subs.append((so, ssz))
            so += ssz
        chunks.append((off, csz, tuple(subs)))
        off += csz

    @functools.partial(
        pl.kernel, mesh=mesh,
        out_type=jax.ShapeDtypeStruct((B, D), jnp.float32),
        compiler_params=pltpu.CompilerParams(use_tc_tiling_on_sc=False),
        scratch_types=[
            pltpu.VMEM((b_per_w,), jnp.int32),
            pltpu.VMEM((512, D), jnp.float32),
            pltpu.SemaphoreType.DMA,
        ],
    )
    def k(table_hbm, idx_hbm, out_hbm, idx_v, rows_v, sem):
        wid = lax.axis_index("s") * 2 + lax.axis_index("c")
        base = wid * b_per_w
        pltpu.sync_copy(idx_hbm.at[pl.ds(base, b_per_w)], idx_v)
        for coff, csz, subs in chunks:
            for so, ssz in subs:
                pltpu.async_copy(
                    table_hbm.at[idx_v.at[pl.ds(coff + so, ssz)]],
                    rows_v.at[pl.ds(so, ssz)], sem).wait()
            pltpu.sync_copy(rows_v.at[pl.ds(0, csz)],
                            out_hbm.at[pl.ds(base + coff, csz)])

    return k


def _gather_rows(table, idx):
    V, D = table.shape
    B = idx.shape[0]
    return _make_sc_gather(V, D, B)(table, idx)


def _segsum(rows, idx, nseg):
    return jax.ops.segment_sum(rows, idx, num_segments=nseg)


@functools.lru_cache(maxsize=None)
def _make_sc_segsum_atoms(E, D, NSEG):
    assert E % _NW == 0 and NSEG % 16 == 0
    e_per_w = E // _NW
    nb, rem = divmod(e_per_w, 128)
    stripe = NSEG // 16
    mesh = plsc.VectorSubcoreMesh(core_axis_name="c", subcore_axis_name="s")

    @functools.partial(
        pl.kernel, mesh=mesh,
        out_type=jax.ShapeDtypeStruct((2 * NSEG, D), jnp.float32),
        compiler_params=pltpu.CompilerParams(use_tc_tiling_on_sc=False),
        scratch_types=[
            pltpu.VMEM_SHARED((NSEG, D), jnp.float32),
            pltpu.VMEM((128,), jnp.int32),
            pltpu.VMEM((8,), jnp.int32),
            pltpu.VMEM((128, D), jnp.float32),
            pltpu.SemaphoreType.DMA,
        ],
    )
    def k(src_hbm, idx_hbm, zero_hbm, out_hbm, shared, idx_v, idxt_v, rows_v, sem):
        c = lax.axis_index("c")
        s = lax.axis_index("s")
        wid = s * 2 + c
        base = wid * e_per_w
        pltpu.sync_copy(zero_hbm, shared.at[pl.ds(s * stripe, stripe)])
        plsc.subcore_barrier()

        def body(j, _):
            off = base + j * 128
            pltpu.sync_copy(idx_hbm.at[pl.ds(off, 128)], idx_v)
            pltpu.sync_copy(src_hbm.at[pl.ds(off, 128)], rows_v)
            pltpu.sync_copy(rows_v, shared.at[idx_v], add=True)
            return _

        lax.fori_loop(0, nb, body, 0)
        if rem:
            off = base + nb * 128
            pltpu.sync_copy(idx_hbm.at[pl.ds(off, rem)], idxt_v)
            pltpu.sync_copy(src_hbm.at[pl.ds(off, rem)], rows_v.at[pl.ds(0, rem)])
            pltpu.sync_copy(rows_v.at[pl.ds(0, rem)], shared.at[idxt_v], add=True)
        plsc.subcore_barrier()
        pltpu.sync_copy(shared.at[pl.ds(s * stripe, stripe)],
                        out_hbm.at[pl.ds(c * NSEG + s * stripe, stripe)])

    return k


_TRI_R = 40000
_TRI_NRANGE = 4


_SB = 512
_NSUB = _SB // 128


@functools.lru_cache(maxsize=None)
def _make_sc_segsum_tri(A, D, NSEG):
    DH = D // 2
    assert A % 16 == 0 and NSEG * 2 == _TRI_R * _TRI_NRANGE * 2
    a_per_t = A // 16
    nb, rem = divmod(a_per_t, _SB)
    assert rem % 16 == 0 and rem <= 128
    zstripe = (_TRI_R + 16) // 16
    ostripe = _TRI_R // 16
    mesh = plsc.VectorSubcoreMesh(core_axis_name="c", subcore_axis_name="s")

    loc_scratch = [pltpu.VMEM((128,), jnp.int32) for _ in range(_NSUB)]

    @functools.partial(
        pl.kernel, mesh=mesh,
        out_type=jax.ShapeDtypeStruct((NSEG, D), jnp.float32),
        compiler_params=pltpu.CompilerParams(use_tc_tiling_on_sc=False),
        scratch_types=[
            pltpu.VMEM_SHARED((_TRI_R + 16, DH), jnp.float32),
            pltpu.VMEM((_SB,), jnp.int32),
            pltpu.VMEM((32,), jnp.int32),
            pltpu.VMEM((_SB, DH), jnp.float32),
        ] + loc_scratch,
    )
    def k(src_hbm, idx_hbm, zero_hbm, out_hbm, shared, idx_v, loct_v, rows_v,
          *locs):
        c = lax.axis_index("c")
        s = lax.axis_index("s")
        tbase = s * a_per_t

        dump = _TRI_R + s

        def localize(n, dst, voff, rng_base):
            for v in range(n // 16):
                iv = idx_v[pl.ds(voff * 128 + v * 16, 16)]
                loc = iv - rng_base
                oob = (loc < 0) | (loc >= _TRI_R)
                dst[pl.ds(v * 16, 16)] = jnp.where(oob, dump, loc)

        for half in range(2):
            cs = half * DH
            for pr in range(_TRI_NRANGE // 2):
                rng_base = (c * (_TRI_NRANGE // 2) + pr) * _TRI_R
                pltpu.sync_copy(zero_hbm,
                                shared.at[pl.ds(s * zstripe, zstripe)])
                plsc.subcore_barrier()

                def body(g, carry):
                    off = tbase + g * _SB
                    pltpu.sync_copy(idx_hbm.at[pl.ds(off, _SB)], idx_v)
                    pltpu.sync_copy(
                        src_hbm.at[pl.ds(off, _SB), pl.ds(cs, DH)], rows_v)
                    for q in range(_NSUB):
                        localize(128, locs[q], q, rng_base)
                        pltpu.sync_copy(rows_v.at[pl.ds(q * 128, 128)],
                                        shared.at[locs[q]], add=True)
                    return carry

                lax.fori_loop(0, nb, body, 0)
                if rem:
                    off = tbase + nb * _SB
                    pltpu.sync_copy(idx_hbm.at[pl.ds(off, rem)],
                                    idx_v.at[pl.ds(0, rem)])
                    localize(rem, loct_v, 0, rng_base)
                    pltpu.sync_copy(
                        src_hbm.at[pl.ds(off, rem), pl.ds(cs, DH)],
                        rows_v.at[pl.ds(0, rem)])
                    pltpu.sync_copy(rows_v.at[pl.ds(0, rem)],
                                    shared.at[loct_v], add=True)
                plsc.subcore_barrier()
                pltpu.sync_copy(
                    shared.at[pl.ds(s * ostripe, ostripe)],
                    out_hbm.at[pl.ds(rng_base + s * ostripe, ostripe),
                               pl.ds(cs, DH)])
                plsc.subcore_barrier()

    return k


def _pad8(w, rows=8):
    out = jnp.zeros((rows,) + w.shape[1:], w.dtype)
    return out.at[:w.shape[0]].set(w)


@jax.jit
def _forward(distances, angles, params, species, idx_i, idx_j, angle_mask,
             reduce_to_ji, expand_to_kj):
    zflat = jnp.asarray(_ZFLAT)
    nflat = jnp.asarray(_NFLAT)
    sel = jnp.asarray(_SEL)
    exp = jnp.asarray(_EXP)
    freq8 = _pad8(params['freq'].reshape(-1, 1), 8).reshape(1, 8)

    x_col = (distances / R_CUTOFF).reshape(-1, 1)
    rbf8, rad48 = _basis(x_col, zflat, nflat, sel, freq8)

    cbf8 = _cbf(angles.reshape(-1, 1),
                angle_mask.astype(jnp.float32).reshape(-1, 1))

    expand_i32 = expand_to_kj.astype(jnp.int32)
    rad_g = _gather_rows(rad48, expand_i32)
    sbf = _sbf48(rad_g, cbf8, exp)

    species_p = jnp.zeros((N_ATOMS_PAD,), jnp.int32).at[:species.shape[0]].set(
        species.astype(jnp.int32))
    h = _gather_rows(params['emb'], species_p)
    hj = _gather_rows(h, idx_j.astype(jnp.int32))
    hi = _gather_rows(h, idx_i.astype(jnp.int32))

    we = params['W_edge']
    m, prod = _edge_embed(
        hj, hi, rbf8,
        _pad8(params['W_rbf_emb']), params['b_rbf_emb'].reshape(1, -1),
        we[:64], we[64:128], we[128:], params['b_edge'].reshape(1, -1),
        _pad8(params['out_blocks'][0]['W_rbf']))

    out_acc = jnp.zeros((N_ATOMS_PAD, 8), jnp.float32)
    idx_i32 = idx_i.astype(jnp.int32)
    reduce_i32 = reduce_to_ji.astype(jnp.int32)
    z_atoms = jnp.zeros((N_ATOMS_PAD // 16, 128), jnp.float32)
    z_tri = jnp.zeros(((_TRI_R + 16) // 16, 32), jnp.float32)
    seg_atoms = _make_sc_segsum_atoms(N_EDGES, 128, N_ATOMS_PAD)
    seg_tri = _make_sc_segsum_tri(N_ANGLES, 64, N_EDGES)

    for i in range(N_INTER + 1):
        pf = seg_atoms(prod, idx_i32, z_atoms)
        out_acc = _out_mlp(pf[:N_ATOMS_PAD], pf[N_ATOMS_PAD:], out_acc,
                           params['out_blocks'][i])
        if i == N_INTER:
            break
        ip = params['int_blocks'][i]
        wrbfc = _pad8(_mm(ip['W_rbf1'], ip['W_rbf2']))
        wc48 = _pad8(_mm(ip['W_sbf1'], ip['W_sbf2']), 48)
        xji, xdown = _inter_a(m, rbf8, ip['W_ji'], ip['b_ji'].reshape(1, -1),
                              ip['W_kj'], ip['b_kj'].reshape(1, -1),
                              wrbfc, ip['W_down'])
        gx = _gather_rows(xdown, expand_i32)
        t = _tri_t(gx, sbf, wc48)
        agg = seg_tri(t, reduce_i32, z_tri)
        m, prod = _inter_b(agg, xji, m, rbf8, ip['W_up'],
                           ip['res_before'][0], ip['W_skip'],
                           ip['b_skip'].reshape(1, -1),
                           ip['res_after'][0], ip['res_after'][1],
                           _pad8(params['out_blocks'][i + 1]['W_rbf']))

    return out_acc[:10000, :1]


def kernel(distances, angles, params, species, idx_i, idx_j, angle_mask,
           reduce_to_ji, expand_to_kj):
    return _forward(distances, angles, params, species, idx_i, idx_j,
                    angle_mask, reduce_to_ji, expand_to_kj)


# speedup vs baseline: 1.4906x; 1.0002x over previous
import functools

import jax
import jax.numpy as jnp
import numpy as np
from jax import lax
from jax.experimental import pallas as pl
from jax.experimental.pallas import tpu as pltpu
from jax.experimental.pallas import tpu_sc as plsc

R_CUTOFF = 5.0
NUM_RBF = 6
NUM_SBF = 7
EMBED = 128
ENV_P = 6
ANGLE_EMB = 64
OUT_EMB = 256
N_INTER = 4

N_EDGES = 160000
N_ANGLES = 320000
N_ATOMS_PAD = 10240

BE = 1000
BA = 1000
BP = 1024


def _sph_jl_np(l, x):
    x = np.asarray(x, dtype=np.float64)
    j0 = np.sin(x) / x
    if l == 0:
        return j0
    j1 = np.sin(x) / x**2 - np.cos(x) / x
    if l == 1:
        return j1
    jm, jc = j0, j1
    for i in range(1, l):
        jn = (2 * i + 1) / x * jc - jm
        jm, jc = jc, jn
    return jc


def _bessel_zeros(num_l, num_n):
    zeros = np.zeros((num_l, num_n))
    xs = np.linspace(1e-2, 80.0, 160001)
    for l in range(num_l):
        vals = _sph_jl_np(l, xs)
        s = np.sign(vals)
        idx = np.where(s[:-1] * s[1:] < 0)[0][:num_n]
        for n, i in enumerate(idx):
            a, b = xs[i], xs[i + 1]
            fa = _sph_jl_np(l, np.array([a]))[0]
            for _ in range(60):
                mid = 0.5 * (a + b)
                fm = _sph_jl_np(l, np.array([mid]))[0]
                if fa * fm <= 0:
                    b = mid
                else:
                    a, fa = mid, fm
            zeros[l, n] = 0.5 * (a + b)
    return zeros


_ZEROS_NP = _bessel_zeros(NUM_SBF, NUM_RBF)
_NORM_NP = np.zeros((NUM_SBF, NUM_RBF))
for _l in range(NUM_SBF):
    _NORM_NP[_l] = np.sqrt(2.0 / R_CUTOFF**3) / np.abs(_sph_jl_np(_l + 1, _ZEROS_NP[_l]))

_LEG_NP = np.sqrt((2 * np.arange(NUM_SBF) + 1) / (4 * np.pi)).astype(np.float32)

_ZFLAT = np.ones((1, 48), np.float32)
_ZFLAT[0, :42] = _ZEROS_NP.reshape(-1).astype(np.float32)
_NFLAT = np.zeros((1, 48), np.float32)
_NFLAT[0, :42] = _NORM_NP.reshape(-1).astype(np.float32)
_SEL = np.zeros((8, 48), np.float32)
for _l in range(NUM_SBF):
    _SEL[_l, _l * 6:(_l + 1) * 6] = 1.0
_EXP = np.zeros((8, 48), np.float32)
for _l in range(NUM_SBF):
    _EXP[_l, _l * 6:(_l + 1) * 6] = 1.0


def _swish(x):
    return x / (1.0 + jnp.exp(-x))


def _mm(a, b):
    return jnp.dot(a, b, preferred_element_type=jnp.float32)


def _wspec(shape):
    nd = len(shape)
    return pl.BlockSpec(shape, lambda i, _n=nd: (0,) * _n)


def _rspec(shape):
    return pl.BlockSpec(shape, lambda i: (i,) + (0,) * (len(shape) - 1))


def _basis_body(x_ref, zflat_ref, nflat_ref, sel_ref, freq_ref, rbf_ref, rad_ref):
    x = x_ref[...]
    arg = x * zflat_ref[...]
    inv = 1.0 / x
    x2 = x * x
    x3 = x2 * x
    x6 = x3 * x3
    x7 = x6 * x
    x8 = x6 * x2
    p = ENV_P + 1
    a = -(p + 1) * (p + 2) / 2.0
    b = p * (p + 2)
    c = -p * (p + 1) / 2.0
    env = jnp.where(x < 1.0, inv + a * x6 + b * x7 + c * x8, 0.0)

    s = jnp.sin(arg)
    co = jnp.cos(arg)
    j0 = s / arg
    j1 = s / (arg * arg) - co / arg
    js = [j0, j1]
    for i in range(1, NUM_SBF - 1):
        js.append((2 * i + 1) / arg * js[i] - js[i - 1])
    sel = jnp.zeros_like(arg)
    for l in range(NUM_SBF):
        sel = sel + js[l] * sel_ref[l:l + 1, :]
    rad_ref[...] = sel * nflat_ref[...] * env
    rbf_ref[...] = env * jnp.sin(freq_ref[...] * x)


def _basis(x_col, zflat, nflat, sel, freq8):
    n = x_col.shape[0]
    return pl.pallas_call(
        _basis_body,
        grid=(n // BE,),
        in_specs=[_rspec((BE, 1)), _wspec((1, 48)), _wspec((1, 48)),
                  _wspec((8, 48)), _wspec((1, 8))],
        out_specs=[_rspec((BE, 8)), _rspec((BE, 48))],
        out_shape=[jax.ShapeDtypeStruct((n, 8), jnp.float32),
                   jax.ShapeDtypeStruct((n, 48), jnp.float32)],
    )(x_col, zflat, nflat, sel, freq8)


def _cbf_body(ang_ref, mask_ref, out_ref):
    ct = jnp.cos(ang_ref[...])
    ps = [jnp.ones_like(ct), ct]
    for l in range(1, NUM_SBF - 1):
        ps.append(((2 * l + 1) * ct * ps[l] - l * ps[l - 1]) / (l + 1))
    msk = mask_ref[...]
    cols = [_LEG_NP[l] * ps[l] * msk for l in range(NUM_SBF)]
    cols.append(jnp.zeros_like(ct))
    out_ref[...] = jnp.concatenate(cols, axis=1)


def _cbf(ang_col, mask_col):
    n = ang_col.shape[0]
    return pl.pallas_call(
        _cbf_body,
        grid=(n // BA,),
        in_specs=[_rspec((BA, 1)), _rspec((BA, 1))],
        out_specs=_rspec((BA, 8)),
        out_shape=jax.ShapeDtypeStruct((n, 8), jnp.float32),
    )(ang_col, mask_col)


def _sbf48_body(radg_ref, cbf_ref, exp_ref, out_ref):
    out_ref[...] = radg_ref[...] * _mm(cbf_ref[...], exp_ref[...])


def _sbf48(rad_g, cbf8, exp):
    n = rad_g.shape[0]
    return pl.pallas_call(
        _sbf48_body,
        grid=(n // BA,),
        in_specs=[_rspec((BA, 48)), _rspec((BA, 8)), _wspec((8, 48))],
        out_specs=_rspec((BA, 48)),
        out_shape=jax.ShapeDtypeStruct((n, 48), jnp.float32),
    )(rad_g, cbf8, exp)


def _edge_embed_body(hj_ref, hi_ref, rbf_ref, wre_ref, bre_ref, wj_ref,
                     wi_ref, wr_ref, be_ref, wrbf0_ref, m_ref, prod_ref):
    rbf = rbf_ref[...]
    rbf_e = _swish(_mm(rbf, wre_ref[...]) + bre_ref[...])
    m = _swish(_mm(hj_ref[...], wj_ref[...]) + _mm(hi_ref[...], wi_ref[...])
               + _mm(rbf_e, wr_ref[...]) + be_ref[...])
    m_ref[...] = m
    prod_ref[...] = _mm(rbf, wrbf0_ref[...]) * m


def _edge_embed(hj, hi, rbf8, wre, bre, wj, wi, wr, be_, wrbf0):
    n = hj.shape[0]
    return pl.pallas_call(
        _edge_embed_body,
        grid=(n // BE,),
        in_specs=[_rspec((BE, 64)), _rspec((BE, 64)), _rspec((BE, 8)),
                  _wspec((8, 128)), _wspec((1, 128)), _wspec((64, 128)),
                  _wspec((64, 128)), _wspec((128, 128)), _wspec((1, 128)),
                  _wspec((8, 128))],
        out_specs=[_rspec((BE, 128)), _rspec((BE, 128))],
        out_shape=[jax.ShapeDtypeStruct((n, 128), jnp.float32),
                   jax.ShapeDtypeStruct((n, 128), jnp.float32)],
    )(hj, hi, rbf8, wre, bre, wj, wi, wr, be_, wrbf0)


def _inter_a_body(m_ref, rbf_ref, wji_ref, bji_ref, wkj_ref, bkj_ref,
                  wrbfc_ref, wdown_ref, xji_ref, xdown_ref):
    m = m_ref[...]
    xji_ref[...] = _swish(_mm(m, wji_ref[...]) + bji_ref[...])
    x_kj = _swish(_mm(m, wkj_ref[...]) + bkj_ref[...])
    x_kj = x_kj * _mm(rbf_ref[...], wrbfc_ref[...])
    xdown_ref[...] = _swish(_mm(x_kj, wdown_ref[...]))


def _inter_a(m, rbf8, wji, bji, wkj, bkj, wrbfc, wdown):
    n = m.shape[0]
    return pl.pallas_call(
        _inter_a_body,
        grid=(n // BE,),
        in_specs=[_rspec((BE, 128)), _rspec((BE, 8)), _wspec((128, 128)),
                  _wspec((1, 128)), _wspec((128, 128)), _wspec((1, 128)),
                  _wspec((8, 128)), _wspec((128, 64))],
        out_specs=[_rspec((BE, 128)), _rspec((BE, 64))],
        out_shape=[jax.ShapeDtypeStruct((n, 128), jnp.float32),
                   jax.ShapeDtypeStruct((n, 64), jnp.float32)],
    )(m, rbf8, wji, bji, wkj, bkj, wrbfc, wdown)


def _tri_t_body(gx_ref, sbf_ref, wc_ref, out_ref):
    out_ref[...] = gx_ref[...] * _mm(sbf_ref[...], wc_ref[...])


def _tri_t(gx, sbf48, wc):
    n = gx.shape[0]
    return pl.pallas_call(
        _tri_t_body,
        grid=(n // BA,),
        in_specs=[_rspec((BA, 64)), _rspec((BA, 48)), _wspec((48, 64))],
        out_specs=_rspec((BA, 64)),
        out_shape=jax.ShapeDtypeStruct((n, 64), jnp.float32),
    )(gx, sbf48, wc)


def _inter_b_body(agg_ref, xji_ref, m_ref, rbf_ref, wup_ref,
                  rb_w1, rb_b1, rb_w2, rb_b2, wskip_ref, bskip_ref,
                  ra1_w1, ra1_b1, ra1_w2, ra1_b2,
                  ra2_w1, ra2_b1, ra2_w2, ra2_b2, wrbfn_ref,
                  mnew_ref, prod_ref):
    x_kj = _swish(_mm(agg_ref[...], wup_ref[...]))
    h = xji_ref[...] + x_kj
    h = h + _swish(_mm(_swish(_mm(h, rb_w1[...]) + rb_b1[...]), rb_w2[...]) + rb_b2[...])
    h = _swish(_mm(h, wskip_ref[...]) + bskip_ref[...]) + m_ref[...]
    h = h + _swish(_mm(_swish(_mm(h, ra1_w1[...]) + ra1_b1[...]), ra1_w2[...]) + ra1_b2[...])
    h = h + _swish(_mm(_swish(_mm(h, ra2_w1[...]) + ra2_b1[...]), ra2_w2[...]) + ra2_b2[...])
    mnew_ref[...] = h
    prod_ref[...] = _mm(rbf_ref[...], wrbfn_ref[...]) * h


def _inter_b(agg, xji, m, rbf8, wup, rb, wskip, bskip, ra1, ra2, wrbfn):
    n = m.shape[0]
    w128 = _wspec((128, 128))
    b128 = _wspec((1, 128))
    return pl.pallas_call(
        _inter_b_body,
        grid=(n // BE,),
        in_specs=[_rspec((BE, 64)), _rspec((BE, 128)), _rspec((BE, 128)),
                  _rspec((BE, 8)), _wspec((64, 128)),
                  w128, b128, w128, b128, w128, b128,
                  w128, b128, w128, b128,
                  w128, b128, w128, b128, _wspec((8, 128))],
        out_specs=[_rspec((BE, 128)), _rspec((BE, 128))],
        out_shape=[jax.ShapeDtypeStruct((n, 128), jnp.float32),
                   jax.ShapeDtypeStruct((n, 128), jnp.float32)],
    )(agg, xji, m, rbf8, wup,
      rb['W1'], rb['b1'].reshape(1, -1), rb['W2'], rb['b2'].reshape(1, -1),
      wskip, bskip,
      ra1['W1'], ra1['b1'].reshape(1, -1), ra1['W2'], ra1['b2'].reshape(1, -1),
      ra2['W1'], ra2['b1'].reshape(1, -1), ra2['W2'], ra2['b2'].reshape(1, -1),
      wrbfn)


def _out_mlp_body(p0_ref, p1_ref, prev_ref, wup_ref, w1, b1, w2, b2, w3, b3,
                  wout_ref, out_ref):
    t = _mm(p0_ref[...] + p1_ref[...], wup_ref[...])
    t = _swish(_mm(t, w1[...]) + b1[...])
    t = _swish(_mm(t, w2[...]) + b2[...])
    t = _swish(_mm(t, w3[...]) + b3[...])
    out_ref[...] = prev_ref[...] + _mm(t, wout_ref[...])


def _out_mlp(p0, p1, prev, ob):
    n = p0.shape[0]
    wout8 = jnp.zeros((OUT_EMB, 8), jnp.float32).at[:, :1].set(ob['W_out'])
    w256 = _wspec((256, 256))
    b256 = _wspec((1, 256))
    return pl.pallas_call(
        _out_mlp_body,
        grid=(n // BP,),
        in_specs=[_rspec((BP, 128)), _rspec((BP, 128)), _rspec((BP, 8)),
                  _wspec((128, 256)), w256, b256, w256, b256, w256, b256,
                  _wspec((256, 8))],
        out_specs=_rspec((BP, 8)),
        out_shape=jax.ShapeDtypeStruct((n, 8), jnp.float32),
    )(p0, p1, prev, ob['W_up'],
      ob['Ws'][0], ob['bs'][0].reshape(1, -1),
      ob['Ws'][1], ob['bs'][1].reshape(1, -1),
      ob['Ws'][2], ob['bs'][2].reshape(1, -1),
      wout8)


_NW = 32


@functools.lru_cache(maxsize=None)
def _make_sc_gather(V, D, B):
    assert D % 16 == 0 and B % (8 * _NW) == 0
    b_per_w = B // _NW
    nb, rem = divmod(b_per_w, 128)
    mesh = plsc.VectorSubcoreMesh(core_axis_name="c", subcore_axis_name="s")

    chunks = []
    off = 0
    while off < b_per_w:
        csz = min(512, b_per_w - off)
        subs = []
        so = 0
        while so < csz:
            ssz = min(128, csz - so)
            subs.append((so, ssz))
            so += ssz
        chunks.append((off, csz, tuple(subs)))
        off += csz

    @functools.partial(
        pl.kernel, mesh=mesh,
        out_type=jax.ShapeDtypeStruct((B, D), jnp.float32),
        compiler_params=pltpu.CompilerParams(use_tc_tiling_on_sc=False),
        scratch_types=[
            pltpu.VMEM((b_per_w,), jnp.int32),
            pltpu.VMEM((512, D), jnp.float32),
            pltpu.SemaphoreType.DMA,
        ],
    )
    def k(table_hbm, idx_hbm, out_hbm, idx_v, rows_v, sem):
        wid = lax.axis_index("s") * 2 + lax.axis_index("c")
        base = wid * b_per_w
        pltpu.sync_copy(idx_hbm.at[pl.ds(base, b_per_w)], idx_v)
        for coff, csz, subs in chunks:
            for so, ssz in subs:
                pltpu.async_copy(
                    table_hbm.at[idx_v.at[pl.ds(coff + so, ssz)]],
                    rows_v.at[pl.ds(so, ssz)], sem).wait()
            pltpu.sync_copy(rows_v.at[pl.ds(0, csz)],
                            out_hbm.at[pl.ds(base + coff, csz)])

    return k


def _gather_rows(table, idx):
    V, D = table.shape
    B = idx.shape[0]
    return _make_sc_gather(V, D, B)(table, idx)


@functools.lru_cache(maxsize=None)
def _make_sc_segsum_atoms(E, D, NSEG):
    assert E % _NW == 0 and NSEG % 16 == 0
    e_per_w = E // _NW
    nb, rem = divmod(e_per_w, 128)
    stripe = NSEG // 16
    mesh = plsc.VectorSubcoreMesh(core_axis_name="c", subcore_axis_name="s")

    @functools.partial(
        pl.kernel, mesh=mesh,
        out_type=jax.ShapeDtypeStruct((2 * NSEG, D), jnp.float32),
        compiler_params=pltpu.CompilerParams(use_tc_tiling_on_sc=False),
        scratch_types=[
            pltpu.VMEM_SHARED((NSEG, D), jnp.float32),
            pltpu.VMEM((128,), jnp.int32),
            pltpu.VMEM((8,), jnp.int32),
            pltpu.VMEM((128, D), jnp.float32),
            pltpu.SemaphoreType.DMA,
        ],
    )
    def k(src_hbm, idx_hbm, zero_hbm, out_hbm, shared, idx_v, idxt_v, rows_v, sem):
        c = lax.axis_index("c")
        s = lax.axis_index("s")
        wid = s * 2 + c
        base = wid * e_per_w
        pltpu.sync_copy(zero_hbm, shared.at[pl.ds(s * stripe, stripe)])
        plsc.subcore_barrier()

        def body(j, _):
            off = base + j * 128
            pltpu.sync_copy(idx_hbm.at[pl.ds(off, 128)], idx_v)
            pltpu.sync_copy(src_hbm.at[pl.ds(off, 128)], rows_v)
            pltpu.sync_copy(rows_v, shared.at[idx_v], add=True)
            return _

        lax.fori_loop(0, nb, body, 0)
        if rem:
            off = base + nb * 128
            pltpu.sync_copy(idx_hbm.at[pl.ds(off, rem)], idxt_v)
            pltpu.sync_copy(src_hbm.at[pl.ds(off, rem)], rows_v.at[pl.ds(0, rem)])
            pltpu.sync_copy(rows_v.at[pl.ds(0, rem)], shared.at[idxt_v], add=True)
        plsc.subcore_barrier()
        pltpu.sync_copy(shared.at[pl.ds(s * stripe, stripe)],
                        out_hbm.at[pl.ds(c * NSEG + s * stripe, stripe)])

    return k


_TRI_R = 40000
_TRI_NRANGE = 4


_SB = 512
_NSUB = _SB // 128


@functools.lru_cache(maxsize=None)
def _make_sc_segsum_tri(A, D, NSEG):
    DH = D // 2
    assert A % 16 == 0 and NSEG * 2 == _TRI_R * _TRI_NRANGE * 2
    a_per_t = A // 16
    nb, rem = divmod(a_per_t, _SB)
    assert rem % 16 == 0 and rem <= 128
    zstripe = (_TRI_R + 16) // 16
    ostripe = _TRI_R // 16
    mesh = plsc.VectorSubcoreMesh(core_axis_name="c", subcore_axis_name="s")

    loc_scratch = [pltpu.VMEM((128,), jnp.int32) for _ in range(_NSUB)]

    @functools.partial(
        pl.kernel, mesh=mesh,
        out_type=jax.ShapeDtypeStruct((NSEG, D), jnp.float32),
        compiler_params=pltpu.CompilerParams(use_tc_tiling_on_sc=False),
        scratch_types=[
            pltpu.VMEM_SHARED((_TRI_R + 16, DH), jnp.float32),
            pltpu.VMEM((_SB,), jnp.int32),
            pltpu.VMEM((32,), jnp.int32),
            pltpu.VMEM((_SB, DH), jnp.float32),
        ] + loc_scratch,
    )
    def k(src_hbm, idx_hbm, zero_hbm, out_hbm, shared, idx_v, loct_v, rows_v,
          *locs):
        c = lax.axis_index("c")
        s = lax.axis_index("s")
        tbase = s * a_per_t

        dump = _TRI_R + s

        def localize(n, dst, voff, rng_base):
            for v in range(n // 16):
                iv = idx_v[pl.ds(voff * 128 + v * 16, 16)]
                loc = iv - rng_base
                oob = (loc < 0) | (loc >= _TRI_R)
                dst[pl.ds(v * 16, 16)] = jnp.where(oob, dump, loc)

        for half in range(2):
            cs = half * DH
            for pr in range(_TRI_NRANGE // 2):
                rng_base = (c * (_TRI_NRANGE // 2) + pr) * _TRI_R
                pltpu.sync_copy(zero_hbm,
                                shared.at[pl.ds(s * zstripe, zstripe)])
                plsc.subcore_barrier()

                def body(g, carry):
                    off = tbase + g * _SB
                    pltpu.sync_copy(idx_hbm.at[pl.ds(off, _SB)], idx_v)
                    pltpu.sync_copy(
                        src_hbm.at[pl.ds(off, _SB), pl.ds(cs, DH)], rows_v)
                    for q in range(_NSUB):
                        localize(128, locs[q], q, rng_base)
                        pltpu.sync_copy(rows_v.at[pl.ds(q * 128, 128)],
                                        shared.at[locs[q]], add=True)
                    return carry

                lax.fori_loop(0, nb, body, 0)
                if rem:
                    off = tbase + nb * _SB
                    pltpu.sync_copy(idx_hbm.at[pl.ds(off, rem)],
                                    idx_v.at[pl.ds(0, rem)])
                    localize(rem, loct_v, 0, rng_base)
                    pltpu.sync_copy(
                        src_hbm.at[pl.ds(off, rem), pl.ds(cs, DH)],
                        rows_v.at[pl.ds(0, rem)])
                    pltpu.sync_copy(rows_v.at[pl.ds(0, rem)],
                                    shared.at[loct_v], add=True)
                plsc.subcore_barrier()
                pltpu.sync_copy(
                    shared.at[pl.ds(s * ostripe, ostripe)],
                    out_hbm.at[pl.ds(rng_base + s * ostripe, ostripe),
                               pl.ds(cs, DH)])
                plsc.subcore_barrier()

    return k


def _pad8(w, rows=8):
    out = jnp.zeros((rows,) + w.shape[1:], w.dtype)
    return out.at[:w.shape[0]].set(w)


@jax.jit
def _forward(distances, angles, params, species, idx_i, idx_j, angle_mask,
             reduce_to_ji, expand_to_kj):
    zflat = jnp.asarray(_ZFLAT)
    nflat = jnp.asarray(_NFLAT)
    sel = jnp.asarray(_SEL)
    exp = jnp.asarray(_EXP)
    freq8 = _pad8(params['freq'].reshape(-1, 1), 8).reshape(1, 8)

    x_col = (distances / R_CUTOFF).reshape(-1, 1)
    rbf8, rad48 = _basis(x_col, zflat, nflat, sel, freq8)

    cbf8 = _cbf(angles.reshape(-1, 1),
                angle_mask.astype(jnp.float32).reshape(-1, 1))

    expand_i32 = expand_to_kj.astype(jnp.int32)
    rad_g = _gather_rows(rad48, expand_i32)
    sbf = _sbf48(rad_g, cbf8, exp)

    species_p = jnp.zeros((N_ATOMS_PAD,), jnp.int32).at[:species.shape[0]].set(
        species.astype(jnp.int32))
    h = _gather_rows(params['emb'], species_p)
    hj = _gather_rows(h, idx_j.astype(jnp.int32))
    hi = _gather_rows(h, idx_i.astype(jnp.int32))

    we = params['W_edge']
    m, prod = _edge_embed(
        hj, hi, rbf8,
        _pad8(params['W_rbf_emb']), params['b_rbf_emb'].reshape(1, -1),
        we[:64], we[64:128], we[128:], params['b_edge'].reshape(1, -1),
        _pad8(params['out_blocks'][0]['W_rbf']))

    out_acc = jnp.zeros((N_ATOMS_PAD, 8), jnp.float32)
    idx_i32 = idx_i.astype(jnp.int32)
    reduce_i32 = reduce_to_ji.astype(jnp.int32)
    z_atoms = jnp.zeros((N_ATOMS_PAD // 16, 128), jnp.float32)
    z_tri = jnp.zeros(((_TRI_R + 16) // 16, 32), jnp.float32)
    seg_atoms = _make_sc_segsum_atoms(N_EDGES, 128, N_ATOMS_PAD)
    seg_tri = _make_sc_segsum_tri(N_ANGLES, 64, N_EDGES)

    for i in range(N_INTER + 1):
        pf = seg_atoms(prod, idx_i32, z_atoms)
        out_acc = _out_mlp(pf[:N_ATOMS_PAD], pf[N_ATOMS_PAD:], out_acc,
                           params['out_blocks'][i])
        if i == N_INTER:
            break
        ip = params['int_blocks'][i]
        wrbfc = _pad8(_mm(ip['W_rbf1'], ip['W_rbf2']))
        wc48 = _pad8(_mm(ip['W_sbf1'], ip['W_sbf2']), 48)
        xji, xdown = _inter_a(m, rbf8, ip['W_ji'], ip['b_ji'].reshape(1, -1),
                              ip['W_kj'], ip['b_kj'].reshape(1, -1),
                              wrbfc, ip['W_down'])
        gx = _gather_rows(xdown, expand_i32)
        t = _tri_t(gx, sbf, wc48)
        agg = seg_tri(t, reduce_i32, z_tri)
        m, prod = _inter_b(agg, xji, m, rbf8, ip['W_up'],
                           ip['res_before'][0], ip['W_skip'],
                           ip['b_skip'].reshape(1, -1),
                           ip['res_after'][0], ip['res_after'][1],
                           _pad8(params['out_blocks'][i + 1]['W_rbf']))

    return out_acc[:10000, :1]


def kernel(distances, angles, params, species, idx_i, idx_j, angle_mask,
           reduce_to_ji, expand_to_kj):
    return _forward(distances, angles, params, species, idx_i, idx_j,
                    angle_mask, reduce_to_ji, expand_to_kj)
